# Initial kernel scaffold; baseline (speedup 1.0000x reference)
import jax
import jax.numpy as jnp
from jax.experimental import pallas as pl


def kernel(x, edge_index, W1_0, W1_1, W1_2, b1_0, b1_1, b1_2, W2_0, W2_1, W2_2, b2_0, b2_1, b2_2, bias_1, bias_2):
    raise NotImplementedError("write your pallas kernel here")


# speedup vs baseline: 2.1585x; 2.1585x over previous
import functools

import jax
import jax.numpy as jnp
from jax import lax
from jax.experimental import pallas as pl
from jax.experimental.pallas import tpu as pltpu
from jax.experimental.pallas import tpu_sc as plsc

N = 10000
NP = 10112
E = 160000
EP = 163840
NC, NS = 2, 16
NW = NC * NS
EW = EP // NS
NB = EW // 128
NBLK = NP // 128


def _matvec_body(nchunks, *refs):
    (gi_h, si_h, zeros_h), vs = refs[0:3], refs[3:3 + nchunks]
    outs = refs[3 + nchunks:3 + 2 * nchunks]
    gcur, scur, rows, acc, sem = refs[3 + 2 * nchunks:]
    cid = lax.axis_index("c")
    sid = lax.axis_index("s")

    for k in range(nchunks):
        @pl.when(cid == k % NC)
        def _chunk(k=k):
            for j in range(5):
                blk = sid + j * NS
                @pl.when(blk < NBLK)
                def _z(blk=blk):
                    pltpu.sync_copy(zeros_h, acc.at[pl.ds(blk * 128, 128)])
            plsc.subcore_barrier()

            def batch(b, _):
                pltpu.sync_copy(gi_h.at[sid, b], gcur)
                pltpu.sync_copy(si_h.at[sid, b], scur)
                pltpu.async_copy(vs[k].at[gcur], rows, sem).wait()
                pltpu.sync_copy(rows, acc.at[scur], add=True)
                return _
            lax.fori_loop(0, NB, batch, None)
            plsc.subcore_barrier()

            for j in range(5):
                blk = sid + j * NS
                @pl.when(blk < NBLK)
                def _f(blk=blk, k=k):
                    pltpu.sync_copy(acc.at[pl.ds(blk * 128, 128)],
                                    outs[k].at[pl.ds(blk * 128, 128)])
            plsc.subcore_barrier()


def _sc_matvec(gidx, sidx, zeros, chunks):
    nchunks = len(chunks)
    mesh = plsc.VectorSubcoreMesh(core_axis_name="c", subcore_axis_name="s")
    fn = pl.kernel(
        functools.partial(_matvec_body, nchunks),
        mesh=mesh,
        out_type=[jax.ShapeDtypeStruct((NP, 128), jnp.float32)] * nchunks,
        scratch_types=[
            pltpu.VMEM((128,), jnp.int32),
            pltpu.VMEM((128,), jnp.int32),
            pltpu.VMEM((128, 128), jnp.float32),
            pltpu.VMEM_SHARED((NP, 128), jnp.float32),
            pltpu.SemaphoreType.DMA,
        ],
        compiler_params=pltpu.CompilerParams(needs_layout_passes=False),
    )
    return fn(gidx, sidx, zeros, *chunks)


def _scatter_sum(gidx, sidx, zeros, vs):
    f = vs.shape[1]
    chunks = [lax.slice_in_dim(vs, 128 * i, 128 * (i + 1), axis=1)
              for i in range(f // 128)]
    outs = _sc_matvec(gidx, sidx, zeros, chunks)
    return jnp.concatenate(list(outs), axis=1)


def _remap_body(r_ref, c_ref, rp_ref, cs_ref):
    r = r_ref[...]
    c = c_ref[...]
    sm = r == c
    rp_ref[...] = jnp.where(sm, N, r)
    cs_ref[...] = jnp.where(sm, N, c)


def _remap(row2d, col2d):
    nb = EP // (128 * 128)
    return pl.pallas_call(
        _remap_body,
        grid=(nb,),
        in_specs=[pl.BlockSpec((128, 128), lambda i: (i, 0))] * 2,
        out_specs=[pl.BlockSpec((128, 128), lambda i: (i, 0))] * 2,
        out_shape=[jax.ShapeDtypeStruct((EP // 128, 128), jnp.int32)] * 2,
    )(row2d, col2d)


def _dis_body(deg_ref, dis_ref):
    d = deg_ref[:, 0:1]
    dis_ref[...] = jnp.where(d > 0, lax.rsqrt(d), 0.0)


def _reduce_dis(deg_s):
    return pl.pallas_call(
        _dis_body,
        grid=(NBLK,),
        in_specs=[pl.BlockSpec((128, 128), lambda i: (i, 0))],
        out_specs=pl.BlockSpec((128, 1), lambda i: (i, 0)),
        out_shape=jax.ShapeDtypeStruct((NP, 1), jnp.float32),
    )(deg_s)


def _ew_call(body, ins, n_out, f):
    blk = lambda i, j: (i, j)
    dspec = pl.BlockSpec((128, 1), lambda i, j: (i, 0))
    specs = [pl.BlockSpec((128, 128), blk) for _ in ins[:-1]] + [dspec]
    return pl.pallas_call(
        body,
        grid=(NBLK, f // 128),
        in_specs=specs,
        out_specs=[pl.BlockSpec((128, 128), blk) for _ in range(n_out)],
        out_shape=[jax.ShapeDtypeStruct((NP, f), jnp.float32)] * n_out,
    )(*ins)


def _scale_body(x, dis, o):
    o[...] = dis[...] * x[...]


def _t1_body(x, s1, dis, t1, t1s):
    d = dis[...]
    t = x[...] - d * s1[...]
    t1[...] = t
    t1s[...] = d * t


def _t2_body(t1, s2, dis, t2):
    t2[...] = t1[...] - dis[...] * s2[...]


def _s_body(u1, u2, s3, dis, s_o, ss_o):
    d = dis[...]
    s = u1[...] + 2.0 * u2[...] - 2.0 * d * s3[...]
    s_o[...] = s
    ss_o[...] = d * s


def _out_body(a, s, s4, dis, o):
    o[...] = a[...] + s[...] - dis[...] * s4[...]


def _mm_body(nk, relu, x_ref, w_ref, b_ref, o_ref):
    k = pl.program_id(2)

    @pl.when(k == 0)
    def _():
        o_ref[...] = jnp.zeros_like(o_ref)

    o_ref[...] += jnp.dot(x_ref[...], w_ref[...],
                          preferred_element_type=jnp.float32)

    @pl.when(k == nk - 1)
    def _():
        y = o_ref[...] + b_ref[...]
        o_ref[...] = jnp.maximum(y, 0.0) if relu else y


def _mm(x, w, b, relu, rb=632, cb=512, kb=256):
    m, kk = x.shape
    _, n = w.shape
    nk = kk // kb
    return pl.pallas_call(
        functools.partial(_mm_body, nk, relu),
        grid=(m // rb, n // cb, nk),
        in_specs=[
            pl.BlockSpec((rb, kb), lambda i, j, k: (i, k)),
            pl.BlockSpec((kb, cb), lambda i, j, k: (k, j)),
            pl.BlockSpec((1, cb), lambda i, j, k: (0, j)),
        ],
        out_specs=pl.BlockSpec((rb, cb), lambda i, j, k: (i, j)),
        out_shape=jax.ShapeDtypeStruct((m, n), jnp.float32),
        compiler_params=pltpu.CompilerParams(
            dimension_semantics=("parallel", "parallel", "arbitrary")),
    )(x, w, b)


def _band(w, in_ch, out_ch):
    li = jnp.arange(128)[:, None]
    lo = jnp.arange(128)[None, :]
    kk = li - lo + 8
    valid = (kk >= 0) & (kk <= 8)
    bm = w[:, :, jnp.clip(kk, 0, 8)]
    bm = jnp.where(valid[None, None], bm, 0.0)
    return bm.transpose(1, 2, 0, 3).reshape(in_ch * 128, out_ch * 128)


def kernel(x, edge_index, W1_0, W1_1, W1_2, b1_0, b1_1, b1_2,
           W2_0, W2_1, W2_2, b2_0, b2_1, b2_2, bias_1, bias_2):
    row = jnp.concatenate([edge_index[0], jnp.full((EP - E,), N, jnp.int32)])
    col = jnp.concatenate([edge_index[1], jnp.full((EP - E,), N, jnp.int32)])

    M0 = _band(W1_0, 2, 16)
    M1 = _band(W1_1, 2, 16)
    M2 = _band(W1_2, 2, 16)
    N0 = _band(W2_0, 16, 8)
    N1 = _band(W2_1, 16, 8)
    N2 = _band(W2_2, 16, 8)
    Mcat = jnp.concatenate([M0 - M2, M1, 2.0 * M2], axis=0)
    Ncat = jnp.concatenate([N0 - N2, N1, N2], axis=1)
    b1f = jnp.repeat(b1_0 + b1_1 + b1_2, 128)[None, :] + bias_1
    b2f = jnp.repeat(b2_0 + b2_1 + b2_2, 128)[None, :] + bias_2
    bcat = jnp.concatenate([b2f, jnp.zeros((1, 2048), jnp.float32)], axis=1)

    xpad = jnp.concatenate([x, jnp.zeros((NP - N, 256), jnp.float32)])
    zeros = jnp.zeros((128, 128), jnp.float32)
    ones_c = jnp.concatenate([jnp.ones((N, 128), jnp.float32),
                              jnp.zeros((NP - N, 128), jnp.float32)])

    rowp2d, colsl2d = _remap(row.reshape(EP // 128, 128),
                             col.reshape(EP // 128, 128))
    rowp = rowp2d.reshape(NS, NB, 128)
    colp = col.reshape(NS, NB, 128)
    colsl = colsl2d.reshape(NS, NB, 128)
    rowr = row.reshape(NS, NB, 128)

    deg_s = _sc_matvec(colsl, rowr, zeros, [ones_c, ones_c])[0]
    dis = _reduce_dis(deg_s)

    xs = _ew_call(_scale_body, [xpad, dis], 1, 256)[0]
    S1 = _scatter_sum(rowp, colp, zeros, xs)
    t1, t1s = _ew_call(_t1_body, [xpad, S1, dis], 2, 256)
    S2 = _scatter_sum(rowp, colp, zeros, t1s)
    t2 = _ew_call(_t2_body, [t1, S2, dis], 1, 256)[0]
    Xcat = jnp.concatenate([xpad, t1, t2], axis=1)
    h = _mm(Xcat, Mcat, b1f, relu=True, kb=256)

    out3 = _mm(h, Ncat, bcat, relu=False, kb=512)
    a = lax.slice_in_dim(out3, 0, 1024, axis=1)
    u1 = lax.slice_in_dim(out3, 1024, 2048, axis=1)
    u2 = lax.slice_in_dim(out3, 2048, 3072, axis=1)
    u2s = _ew_call(_scale_body, [u2, dis], 1, 1024)[0]
    S3 = _scatter_sum(rowp, colp, zeros, u2s)
    s, ss = _ew_call(_s_body, [u1, u2, S3, dis], 2, 1024)
    S4 = _scatter_sum(rowp, colp, zeros, ss)
    out = _ew_call(_out_body, [a, s, S4, dis], 1, 1024)[0]
    return out[:N]


# speedup vs baseline: 2.6503x; 1.2279x over previous
import functools

import jax
import jax.numpy as jnp
from jax import lax
from jax.experimental import pallas as pl
from jax.experimental.pallas import tpu as pltpu
from jax.experimental.pallas import tpu_sc as plsc

N = 10000
NP = 10112
E = 160000
EP = 163840
NC, NS = 2, 16
NW = NC * NS
EW = EP // NS
NB = EW // 128
NBLK = NP // 128


def _matvec_body(nchunks, *refs):
    (ix_h, zeros_h), vs = refs[0:2], refs[2:2 + nchunks]
    outs = refs[2 + nchunks:2 + 2 * nchunks]
    ib0, ib1, rows0, rows1, acc, is0, is1, gs0, gs1 = refs[2 + 2 * nchunks:]
    cid = lax.axis_index("c")
    sid = lax.axis_index("s")

    def idx_start(b, ib, sem):
        pltpu.async_copy(ix_h.at[sid, b], ib, sem)

    def idx_wait(b, ib, sem):
        pltpu.make_async_copy(ix_h.at[sid, b], ib, sem).wait()

    for k in range(nchunks):
        @pl.when(cid == k % NC)
        def _chunk(k=k):
            for j in range(5):
                blk = sid + j * NS
                @pl.when(blk < NBLK)
                def _z(blk=blk):
                    pltpu.sync_copy(zeros_h, acc.at[pl.ds(blk * 128, 128)])
            plsc.subcore_barrier()

            def step(b, ib_c, rows_c, is_c, gs_c, ib_n, rows_n, is_n, gs_n,
                     k=k):
                @pl.when(b + 1 < NB)
                def _():
                    idx_wait(b + 1, ib_n, is_n)
                    pltpu.async_copy(vs[k].at[ib_n.at[0]], rows_n, gs_n)
                pltpu.make_async_copy(vs[k].at[ib_c.at[0]], rows_c,
                                      gs_c).wait()
                pltpu.sync_copy(rows_c, acc.at[ib_c.at[1]], add=True)
                @pl.when(b + 2 < NB)
                def _():
                    idx_start(b + 2, ib_c, is_c)

            idx_start(0, ib0, is0)
            idx_wait(0, ib0, is0)
            pltpu.async_copy(vs[k].at[ib0.at[0]], rows0, gs0)
            idx_start(1, ib1, is1)

            def pair(i, _):
                b = 2 * i
                step(b, ib0, rows0, is0, gs0, ib1, rows1, is1, gs1)
                step(b + 1, ib1, rows1, is1, gs1, ib0, rows0, is0, gs0)
                return _
            lax.fori_loop(0, NB // 2, pair, None)
            plsc.subcore_barrier()

            for j in range(5):
                blk = sid + j * NS
                @pl.when(blk < NBLK)
                def _f(blk=blk, k=k):
                    pltpu.sync_copy(acc.at[pl.ds(blk * 128, 128)],
                                    outs[k].at[pl.ds(blk * 128, 128)])
            plsc.subcore_barrier()


def _sc_matvec(idx_pairs, zeros, chunks):
    nchunks = len(chunks)
    mesh = plsc.VectorSubcoreMesh(core_axis_name="c", subcore_axis_name="s")
    fn = pl.kernel(
        functools.partial(_matvec_body, nchunks),
        mesh=mesh,
        out_type=[jax.ShapeDtypeStruct((NP, 128), jnp.float32)] * nchunks,
        scratch_types=[
            pltpu.VMEM((2, 128), jnp.int32),
            pltpu.VMEM((2, 128), jnp.int32),
            pltpu.VMEM((128, 128), jnp.float32),
            pltpu.VMEM((128, 128), jnp.float32),
            pltpu.VMEM_SHARED((NP, 128), jnp.float32),
            pltpu.SemaphoreType.DMA,
            pltpu.SemaphoreType.DMA,
            pltpu.SemaphoreType.DMA,
            pltpu.SemaphoreType.DMA,
        ],
        compiler_params=pltpu.CompilerParams(needs_layout_passes=False),
    )
    return fn(idx_pairs, zeros, *chunks)


def _scatter_sum(idx_pairs, zeros, vs):
    f = vs.shape[1]
    chunks = [lax.slice_in_dim(vs, 128 * i, 128 * (i + 1), axis=1)
              for i in range(f // 128)]
    outs = _sc_matvec(idx_pairs, zeros, chunks)
    return jnp.concatenate(list(outs), axis=1)


def _remap_body(r_ref, c_ref, rp_ref, cs_ref):
    r = r_ref[...]
    c = c_ref[...]
    sm = r == c
    rp_ref[...] = jnp.where(sm, N, r)
    cs_ref[...] = jnp.where(sm, N, c)


def _remap(row2d, col2d):
    nb = EP // (128 * 128)
    return pl.pallas_call(
        _remap_body,
        grid=(nb,),
        in_specs=[pl.BlockSpec((128, 128), lambda i: (i, 0))] * 2,
        out_specs=[pl.BlockSpec((128, 128), lambda i: (i, 0))] * 2,
        out_shape=[jax.ShapeDtypeStruct((EP // 128, 128), jnp.int32)] * 2,
    )(row2d, col2d)


def _dis_body(deg_ref, dis_ref):
    d = deg_ref[:, 0:1]
    dis_ref[...] = jnp.where(d > 0, lax.rsqrt(d), 0.0)


def _reduce_dis(deg_s):
    return pl.pallas_call(
        _dis_body,
        grid=(NBLK,),
        in_specs=[pl.BlockSpec((128, 128), lambda i: (i, 0))],
        out_specs=pl.BlockSpec((128, 1), lambda i: (i, 0)),
        out_shape=jax.ShapeDtypeStruct((NP, 1), jnp.float32),
    )(deg_s)


def _ew_call(body, ins, n_out, f):
    blk = lambda i, j: (i, j)
    dspec = pl.BlockSpec((128, 1), lambda i, j: (i, 0))
    specs = [pl.BlockSpec((128, 128), blk) for _ in ins[:-1]] + [dspec]
    return pl.pallas_call(
        body,
        grid=(NBLK, f // 128),
        in_specs=specs,
        out_specs=[pl.BlockSpec((128, 128), blk) for _ in range(n_out)],
        out_shape=[jax.ShapeDtypeStruct((NP, f), jnp.float32)] * n_out,
    )(*ins)


def _scale_body(x, dis, o):
    o[...] = dis[...] * x[...]


def _t1_body(x, s1, dis, t1, t1s):
    d = dis[...]
    t = x[...] - d * s1[...]
    t1[...] = t
    t1s[...] = d * t


def _t2_body(t1, s2, dis, t2):
    t2[...] = t1[...] - dis[...] * s2[...]


def _s_body(u1, u2, s3, dis, s_o, ss_o):
    d = dis[...]
    s = u1[...] + 2.0 * u2[...] - 2.0 * d * s3[...]
    s_o[...] = s
    ss_o[...] = d * s


def _out_body(a, s, s4, dis, o):
    o[...] = a[...] + s[...] - dis[...] * s4[...]


def _mm_body(nk, relu, x_ref, w_ref, b_ref, o_ref):
    k = pl.program_id(2)

    @pl.when(k == 0)
    def _():
        o_ref[...] = jnp.zeros_like(o_ref)

    o_ref[...] += jnp.dot(x_ref[...], w_ref[...],
                          preferred_element_type=jnp.float32)

    @pl.when(k == nk - 1)
    def _():
        y = o_ref[...] + b_ref[...]
        o_ref[...] = jnp.maximum(y, 0.0) if relu else y


def _mm(x, w, b, relu, rb=1264, cb=512, kb=256):
    m, kk = x.shape
    _, n = w.shape
    nk = kk // kb
    return pl.pallas_call(
        functools.partial(_mm_body, nk, relu),
        grid=(m // rb, n // cb, nk),
        in_specs=[
            pl.BlockSpec((rb, kb), lambda i, j, k: (i, k)),
            pl.BlockSpec((kb, cb), lambda i, j, k: (k, j)),
            pl.BlockSpec((1, cb), lambda i, j, k: (0, j)),
        ],
        out_specs=pl.BlockSpec((rb, cb), lambda i, j, k: (i, j)),
        out_shape=jax.ShapeDtypeStruct((m, n), jnp.float32),
        compiler_params=pltpu.CompilerParams(
            dimension_semantics=("parallel", "parallel", "arbitrary")),
    )(x, w, b)


def _band(w, in_ch, out_ch):
    li = jnp.arange(128)[:, None]
    lo = jnp.arange(128)[None, :]
    kk = li - lo + 8
    valid = (kk >= 0) & (kk <= 8)
    bm = w[:, :, jnp.clip(kk, 0, 8)]
    bm = jnp.where(valid[None, None], bm, 0.0)
    return bm.transpose(1, 2, 0, 3).reshape(in_ch * 128, out_ch * 128)


def kernel(x, edge_index, W1_0, W1_1, W1_2, b1_0, b1_1, b1_2,
           W2_0, W2_1, W2_2, b2_0, b2_1, b2_2, bias_1, bias_2):
    row = jnp.concatenate([edge_index[0], jnp.full((EP - E,), N, jnp.int32)])
    col = jnp.concatenate([edge_index[1], jnp.full((EP - E,), N, jnp.int32)])

    M0 = _band(W1_0, 2, 16)
    M1 = _band(W1_1, 2, 16)
    M2 = _band(W1_2, 2, 16)
    N0 = _band(W2_0, 16, 8)
    N1 = _band(W2_1, 16, 8)
    N2 = _band(W2_2, 16, 8)
    Mcat = jnp.concatenate([M0 - M2, M1, 2.0 * M2], axis=0)
    Ncat = jnp.concatenate([N0 - N2, N1, N2], axis=1)
    b1f = jnp.repeat(b1_0 + b1_1 + b1_2, 128)[None, :] + bias_1
    b2f = jnp.repeat(b2_0 + b2_1 + b2_2, 128)[None, :] + bias_2
    bcat = jnp.concatenate([b2f, jnp.zeros((1, 2048), jnp.float32)], axis=1)

    xpad = jnp.concatenate([x, jnp.zeros((NP - N, 256), jnp.float32)])
    zeros = jnp.zeros((128, 128), jnp.float32)
    ones_c = jnp.concatenate([jnp.ones((N, 128), jnp.float32),
                              jnp.zeros((NP - N, 128), jnp.float32)])

    rowp2d, colsl2d = _remap(row.reshape(EP // 128, 128),
                             col.reshape(EP // 128, 128))
    ix_main = jnp.stack([rowp2d.reshape(NS, NB, 128),
                         col.reshape(NS, NB, 128)], axis=2)
    ix_deg = jnp.stack([colsl2d.reshape(NS, NB, 128),
                        row.reshape(NS, NB, 128)], axis=2)

    deg_s = _sc_matvec(ix_deg, zeros, [ones_c, ones_c])[0]
    dis = _reduce_dis(deg_s)

    xs = _ew_call(_scale_body, [xpad, dis], 1, 256)[0]
    S1 = _scatter_sum(ix_main, zeros, xs)
    t1, t1s = _ew_call(_t1_body, [xpad, S1, dis], 2, 256)
    S2 = _scatter_sum(ix_main, zeros, t1s)
    t2 = _ew_call(_t2_body, [t1, S2, dis], 1, 256)[0]
    Xcat = jnp.concatenate([xpad, t1, t2], axis=1)
    h = _mm(Xcat.astype(jnp.bfloat16), Mcat.astype(jnp.bfloat16),
            b1f, relu=True, kb=256)

    out3 = _mm(h.astype(jnp.bfloat16), Ncat.astype(jnp.bfloat16),
               bcat, relu=False, kb=512)
    a = lax.slice_in_dim(out3, 0, 1024, axis=1)
    u1 = lax.slice_in_dim(out3, 1024, 2048, axis=1)
    u2 = lax.slice_in_dim(out3, 2048, 3072, axis=1)
    u2s = _ew_call(_scale_body, [u2, dis], 1, 1024)[0]
    S3 = _scatter_sum(ix_main, zeros, u2s)
    s, ss = _ew_call(_s_body, [u1, u2, S3, dis], 2, 1024)
    S4 = _scatter_sum(ix_main, zeros, ss)
    out = _ew_call(_out_body, [a, s, S4, dis], 1, 1024)[0]
    return out[:N]


# speedup vs baseline: 2.8094x; 1.0600x over previous
import functools

import jax
import jax.numpy as jnp
from jax import lax
from jax.experimental import pallas as pl
from jax.experimental.pallas import tpu as pltpu
from jax.experimental.pallas import tpu_sc as plsc

N = 10000
NP = 10112
E = 160000
EP = 163840
NC, NS = 2, 16
NW = NC * NS
EW = EP // NS
NB = EW // 128
NBLK = NP // 128


def _matvec_body(nchunks, deg_mode, *refs):
    nv = 0 if deg_mode else nchunks
    (ix_h, const_h), vs = refs[0:2], refs[2:2 + nv]
    outs = refs[2 + nv:2 + nv + nchunks]
    rest = refs[2 + nv + nchunks:]
    ibs = rest[0:4]
    rowss = rest[4:6]
    acc = rest[6]
    isems = rest[7:11]
    gsems = rest[11:13]
    ssems = rest[13:15]
    cid = lax.axis_index("c")
    sid = lax.axis_index("s")

    def idx_fire(b, q):
        pltpu.async_copy(ix_h.at[sid, b], ibs[q], isems[q])

    def idx_wait(b, q):
        pltpu.make_async_copy(ix_h.at[sid, b], ibs[q], isems[q]).wait()

    def g_fire(k, p, q):
        pltpu.async_copy(vs[k].at[ibs[q].at[0]], rowss[p], gsems[p])

    def g_wait(k, p, q):
        pltpu.make_async_copy(vs[k].at[ibs[q].at[0]], rowss[p],
                              gsems[p]).wait()

    def s_fire(p, q):
        pltpu.async_copy(rowss[p], acc.at[ibs[q].at[1]], ssems[p], add=True)

    def s_wait(p, q):
        pltpu.make_async_copy(rowss[p], acc.at[ibs[q].at[1]],
                              ssems[p]).wait()

    for k in range(nchunks):
        @pl.when(cid == k % NC)
        def _chunk(k=k):
            for j in range(5):
                blk = sid + j * NS
                @pl.when(blk < NBLK)
                def _z(blk=blk):
                    pltpu.sync_copy(const_h.at[0], acc.at[pl.ds(blk * 128, 128)])
            plsc.subcore_barrier()

            if deg_mode:
                pltpu.sync_copy(const_h.at[1], rowss[0])
                idx_fire(0, 0)
                idx_fire(1, 1)

                def quad(i, _):
                    for pos in range(4):
                        b = 4 * i + pos
                        p, q = pos % 2, pos
                        @pl.when(b >= 2)
                        def _(p=p, q=q):
                            s_wait(p, (q + 2) % 4)
                        @pl.when(b + 2 < NB)
                        def _(b=b, q=q):
                            idx_fire(b + 2, (q + 2) % 4)
                        idx_wait(b, q)
                        pltpu.async_copy(rowss[0], acc.at[ibs[q].at[1]],
                                         ssems[p], add=True)
                    return _
                lax.fori_loop(0, NB // 4, quad, None)
                s_wait(0, 2)
                s_wait(1, 3)
            else:
                idx_fire(0, 0)
                idx_fire(1, 1)
                idx_fire(2, 2)
                idx_wait(0, 0)
                g_fire(k, 0, 0)

                def quad(i, _):
                    for pos in range(4):
                        b = 4 * i + pos
                        p, q = pos % 2, pos
                        pn, qn = (pos + 1) % 2, (pos + 1) % 4
                        @pl.when(b >= 1)
                        def _(pn=pn, q=q):
                            s_wait(pn, (q + 3) % 4)
                        @pl.when(b + 3 < NB)
                        def _(b=b, q=q):
                            idx_fire(b + 3, (q + 3) % 4)
                        @pl.when(b + 1 < NB)
                        def _(b=b, pn=pn, qn=qn, k=k):
                            idx_wait(b + 1, qn)
                            g_fire(k, pn, qn)
                        g_wait(k, p, q)
                        s_fire(p, q)
                    return _
                lax.fori_loop(0, NB // 4, quad, None)
                s_wait(1, 3)
            plsc.subcore_barrier()

            for j in range(5):
                blk = sid + j * NS
                @pl.when(blk < NBLK)
                def _f(blk=blk, k=k):
                    pltpu.sync_copy(acc.at[pl.ds(blk * 128, 128)],
                                    outs[k].at[pl.ds(blk * 128, 128)])
            plsc.subcore_barrier()


def _sc_matvec(idx_pairs, const, chunks, deg_mode=False):
    nchunks = 2 if deg_mode else len(chunks)
    mesh = plsc.VectorSubcoreMesh(core_axis_name="c", subcore_axis_name="s")
    fn = pl.kernel(
        functools.partial(_matvec_body, nchunks, deg_mode),
        mesh=mesh,
        out_type=[jax.ShapeDtypeStruct((NP, 128), jnp.float32)] * nchunks,
        scratch_types=[
            pltpu.VMEM((2, 128), jnp.int32),
            pltpu.VMEM((2, 128), jnp.int32),
            pltpu.VMEM((2, 128), jnp.int32),
            pltpu.VMEM((2, 128), jnp.int32),
            pltpu.VMEM((128, 128), jnp.float32),
            pltpu.VMEM((128, 128), jnp.float32),
            pltpu.VMEM_SHARED((NP, 128), jnp.float32),
            pltpu.SemaphoreType.DMA,
            pltpu.SemaphoreType.DMA,
            pltpu.SemaphoreType.DMA,
            pltpu.SemaphoreType.DMA,
            pltpu.SemaphoreType.DMA,
            pltpu.SemaphoreType.DMA,
            pltpu.SemaphoreType.DMA,
            pltpu.SemaphoreType.DMA,
        ],
        compiler_params=pltpu.CompilerParams(needs_layout_passes=False),
    )
    return fn(idx_pairs, const, *chunks)


def _scatter_sum(idx_pairs, zeros, vs):
    f = vs.shape[1]
    chunks = [lax.slice_in_dim(vs, 128 * i, 128 * (i + 1), axis=1)
              for i in range(f // 128)]
    outs = _sc_matvec(idx_pairs, zeros, chunks)
    return jnp.concatenate(list(outs), axis=1)


def _remap_body(r_ref, c_ref, rp_ref, rd_ref):
    r = r_ref[...]
    c = c_ref[...]
    sm = r == c
    rp_ref[...] = jnp.where(sm, N, r)
    rd_ref[...] = jnp.where(sm, N + 8, r)


def _remap(row2d, col2d):
    nb = EP // (128 * 128)
    return pl.pallas_call(
        _remap_body,
        grid=(nb,),
        in_specs=[pl.BlockSpec((128, 128), lambda i: (i, 0))] * 2,
        out_specs=[pl.BlockSpec((128, 128), lambda i: (i, 0))] * 2,
        out_shape=[jax.ShapeDtypeStruct((EP // 128, 128), jnp.int32)] * 2,
    )(row2d, col2d)


def _dis_body(deg_ref, dis_ref):
    d = deg_ref[:, 0:1]
    dis_ref[...] = jnp.where(d > 0, lax.rsqrt(d), 0.0)


def _reduce_dis(deg_s):
    return pl.pallas_call(
        _dis_body,
        grid=(NBLK,),
        in_specs=[pl.BlockSpec((128, 128), lambda i: (i, 0))],
        out_specs=pl.BlockSpec((128, 1), lambda i: (i, 0)),
        out_shape=jax.ShapeDtypeStruct((NP, 1), jnp.float32),
    )(deg_s)


def _ew_call(body, ins, n_out, f):
    blk = lambda i, j: (i, j)
    dspec = pl.BlockSpec((128, 1), lambda i, j: (i, 0))
    specs = [pl.BlockSpec((128, 128), blk) for _ in ins[:-1]] + [dspec]
    return pl.pallas_call(
        body,
        grid=(NBLK, f // 128),
        in_specs=specs,
        out_specs=[pl.BlockSpec((128, 128), blk) for _ in range(n_out)],
        out_shape=[jax.ShapeDtypeStruct((NP, f), jnp.float32)] * n_out,
    )(*ins)


def _scale_body(x, dis, o):
    o[...] = dis[...] * x[...]


def _t1_body(x, s1, dis, t1, t1s):
    d = dis[...]
    t = x[...] - d * s1[...]
    t1[...] = t
    t1s[...] = d * t


def _t2_body(t1, s2, dis, t2):
    t2[...] = t1[...] - dis[...] * s2[...]


def _s_body(u1, u2, s3, dis, s_o, ss_o):
    d = dis[...]
    s = u1[...] + 2.0 * u2[...] - 2.0 * d * s3[...]
    s_o[...] = s
    ss_o[...] = d * s


def _out_body(a, s, s4, dis, o):
    o[...] = a[...] + s[...] - dis[...] * s4[...]


def _mm1_body(x0, x1, x2, w0, w1, w2, b_ref, o_ref):
    acc = jnp.dot(x0[...].astype(jnp.bfloat16), w0[...],
                  preferred_element_type=jnp.float32)
    acc += jnp.dot(x1[...].astype(jnp.bfloat16), w1[...],
                   preferred_element_type=jnp.float32)
    acc += jnp.dot(x2[...].astype(jnp.bfloat16), w2[...],
                   preferred_element_type=jnp.float32)
    o_ref[...] = jnp.maximum(acc + b_ref[...], 0.0)


def _mm1(x0, x1, x2, w0, w1, w2, b, rb=1264, cb=512):
    n = w0.shape[1]
    xspec = pl.BlockSpec((rb, 256), lambda i, j: (i, 0))
    wspec = pl.BlockSpec((256, cb), lambda i, j: (0, j))
    return pl.pallas_call(
        _mm1_body,
        grid=(NP // rb, n // cb),
        in_specs=[xspec, xspec, xspec, wspec, wspec, wspec,
                  pl.BlockSpec((1, cb), lambda i, j: (0, j))],
        out_specs=pl.BlockSpec((rb, cb), lambda i, j: (i, j)),
        out_shape=jax.ShapeDtypeStruct((NP, n), jnp.float32),
        compiler_params=pltpu.CompilerParams(
            dimension_semantics=("parallel", "parallel")),
    )(x0, x1, x2, w0, w1, w2, b)


def _mm_body(nk, x_ref, w_ref, b_ref, o_ref):
    k = pl.program_id(2)

    @pl.when(k == 0)
    def _():
        o_ref[...] = jnp.zeros_like(o_ref)

    o_ref[...] += jnp.dot(x_ref[...].astype(jnp.bfloat16), w_ref[...],
                          preferred_element_type=jnp.float32)

    @pl.when(k == nk - 1)
    def _():
        o_ref[...] += b_ref[...]


def _mm(x, w, b, rb=1264, cb=512, kb=512):
    m, kk = x.shape
    _, n = w.shape
    nk = kk // kb
    return pl.pallas_call(
        functools.partial(_mm_body, nk),
        grid=(m // rb, n // cb, nk),
        in_specs=[
            pl.BlockSpec((rb, kb), lambda i, j, k: (i, k)),
            pl.BlockSpec((kb, cb), lambda i, j, k: (k, j)),
            pl.BlockSpec((1, cb), lambda i, j, k: (0, j)),
        ],
        out_specs=pl.BlockSpec((rb, cb), lambda i, j, k: (i, j)),
        out_shape=jax.ShapeDtypeStruct((m, n), jnp.float32),
        compiler_params=pltpu.CompilerParams(
            dimension_semantics=("parallel", "parallel", "arbitrary")),
    )(x, w, b)


def _band(w, in_ch, out_ch):
    li = jnp.arange(128)[:, None]
    lo = jnp.arange(128)[None, :]
    kk = li - lo + 8
    valid = (kk >= 0) & (kk <= 8)
    bm = w[:, :, jnp.clip(kk, 0, 8)]
    bm = jnp.where(valid[None, None], bm, 0.0)
    return bm.transpose(1, 2, 0, 3).reshape(in_ch * 128, out_ch * 128)


def kernel(x, edge_index, W1_0, W1_1, W1_2, b1_0, b1_1, b1_2,
           W2_0, W2_1, W2_2, b2_0, b2_1, b2_2, bias_1, bias_2):
    row = jnp.concatenate([edge_index[0], jnp.full((EP - E,), N, jnp.int32)])
    col = jnp.concatenate([edge_index[1], jnp.full((EP - E,), N, jnp.int32)])

    M0 = _band(W1_0, 2, 16)
    M1 = _band(W1_1, 2, 16)
    M2 = _band(W1_2, 2, 16)
    N0 = _band(W2_0, 16, 8)
    N1 = _band(W2_1, 16, 8)
    N2 = _band(W2_2, 16, 8)
    Ncat = jnp.concatenate([N0 - N2, N1, N2], axis=1)
    b1f = jnp.repeat(b1_0 + b1_1 + b1_2, 128)[None, :] + bias_1
    b2f = jnp.repeat(b2_0 + b2_1 + b2_2, 128)[None, :] + bias_2
    bcat = jnp.concatenate([b2f, jnp.zeros((1, 2048), jnp.float32)], axis=1)

    xpad = jnp.concatenate([x, jnp.zeros((NP - N, 256), jnp.float32)])
    const = jnp.stack([jnp.zeros((128, 128), jnp.float32),
                       jnp.ones((128, 128), jnp.float32)])

    rowp2d, rdeg2d = _remap(row.reshape(EP // 128, 128),
                            col.reshape(EP // 128, 128))
    ix_main = jnp.stack([rowp2d.reshape(NS, NB, 128),
                         col.reshape(NS, NB, 128)], axis=2)
    rdeg3 = rdeg2d.reshape(NS, NB, 128)
    ix_deg = jnp.stack([rdeg3, rdeg3], axis=2)

    deg_s = _sc_matvec(ix_deg, const, [], deg_mode=True)[0]
    dis = _reduce_dis(deg_s)

    xs = _ew_call(_scale_body, [xpad, dis], 1, 256)[0]
    S1 = _scatter_sum(ix_main, const, xs)
    t1, t1s = _ew_call(_t1_body, [xpad, S1, dis], 2, 256)
    S2 = _scatter_sum(ix_main, const, t1s)
    t2 = _ew_call(_t2_body, [t1, S2, dis], 1, 256)[0]
    bf = jnp.bfloat16
    h = _mm1(xpad, t1, t2, (M0 - M2).astype(bf), M1.astype(bf),
             (2.0 * M2).astype(bf), b1f)

    out3 = _mm(h, Ncat.astype(bf), bcat)
    a = lax.slice_in_dim(out3, 0, 1024, axis=1)
    u1 = lax.slice_in_dim(out3, 1024, 2048, axis=1)
    u2 = lax.slice_in_dim(out3, 2048, 3072, axis=1)
    u2s = _ew_call(_scale_body, [u2, dis], 1, 1024)[0]
    S3 = _scatter_sum(ix_main, const, u2s)
    s, ss = _ew_call(_s_body, [u1, u2, S3, dis], 2, 1024)
    S4 = _scatter_sum(ix_main, const, ss)
    out = _ew_call(_out_body, [a, s, S4, dis], 1, 1024)[0]
    return out[:N]


# speedup vs baseline: 3.0705x; 1.0929x over previous
import functools

import jax
import jax.numpy as jnp
from jax import lax
from jax.experimental import pallas as pl
from jax.experimental.pallas import tpu as pltpu
from jax.experimental.pallas import tpu_sc as plsc

N = 10000
NP = 10112
E = 160000
EP = 163840
NC, NS = 2, 16
NW = NC * NS
EW = EP // NS
NB = EW // 128
NBLK = NP // 128


def _matvec_body(nchunks, deg_mode, *refs):
    nv = 0 if deg_mode else nchunks
    (ix_h, const_h), vs = refs[0:2], refs[2:2 + nv]
    outs = refs[2 + nv:2 + nv + nchunks]
    rest = refs[2 + nv + nchunks:]
    ibs = rest[0:4]
    rowss = rest[4:6]
    acc = rest[6]
    isems = rest[7:11]
    gsems = rest[11:13]
    ssems = rest[13:15]
    cid = lax.axis_index("c")
    sid = lax.axis_index("s")

    def idx_fire(b, q):
        pltpu.async_copy(ix_h.at[sid, b], ibs[q], isems[q])

    def idx_wait(b, q):
        pltpu.make_async_copy(ix_h.at[sid, b], ibs[q], isems[q]).wait()

    def g_fire(k, p, q):
        pltpu.async_copy(vs[k].at[ibs[q].at[0]], rowss[p], gsems[p])

    def g_wait(k, p, q):
        pltpu.make_async_copy(vs[k].at[ibs[q].at[0]], rowss[p],
                              gsems[p]).wait()

    def s_fire(p, q):
        pltpu.async_copy(rowss[p], acc.at[ibs[q].at[1]], ssems[p], add=True)

    def s_wait(p, q):
        pltpu.make_async_copy(rowss[p], acc.at[ibs[q].at[1]],
                              ssems[p]).wait()

    for k in range(nchunks):
        @pl.when(cid == k % NC)
        def _chunk(k=k):
            for j in range(5):
                blk = sid + j * NS
                @pl.when(blk < NBLK)
                def _z(blk=blk):
                    pltpu.sync_copy(const_h.at[0], acc.at[pl.ds(blk * 128, 128)])
            plsc.subcore_barrier()

            if deg_mode:
                pltpu.sync_copy(const_h.at[1], rowss[0])
                idx_fire(0, 0)
                idx_fire(1, 1)

                def quad(i, _):
                    for pos in range(4):
                        b = 4 * i + pos
                        p, q = pos % 2, pos
                        @pl.when(b >= 2)
                        def _(p=p, q=q):
                            s_wait(p, (q + 2) % 4)
                        @pl.when(b + 2 < NB)
                        def _(b=b, q=q):
                            idx_fire(b + 2, (q + 2) % 4)
                        idx_wait(b, q)
                        pltpu.async_copy(rowss[0], acc.at[ibs[q].at[1]],
                                         ssems[p], add=True)
                    return _
                lax.fori_loop(0, NB // 4, quad, None)
                s_wait(0, 2)
                s_wait(1, 3)
            else:
                idx_fire(0, 0)
                idx_fire(1, 1)
                idx_fire(2, 2)
                idx_wait(0, 0)
                g_fire(k, 0, 0)

                def quad(i, _):
                    for pos in range(4):
                        b = 4 * i + pos
                        p, q = pos % 2, pos
                        pn, qn = (pos + 1) % 2, (pos + 1) % 4
                        @pl.when(b >= 1)
                        def _(pn=pn, q=q):
                            s_wait(pn, (q + 3) % 4)
                        @pl.when(b + 3 < NB)
                        def _(b=b, q=q):
                            idx_fire(b + 3, (q + 3) % 4)
                        @pl.when(b + 1 < NB)
                        def _(b=b, pn=pn, qn=qn, k=k):
                            idx_wait(b + 1, qn)
                            g_fire(k, pn, qn)
                        g_wait(k, p, q)
                        s_fire(p, q)
                    return _
                lax.fori_loop(0, NB // 4, quad, None)
                s_wait(1, 3)
            plsc.subcore_barrier()

            for j in range(5):
                blk = sid + j * NS
                @pl.when(blk < NBLK)
                def _f(blk=blk, k=k):
                    pltpu.sync_copy(acc.at[pl.ds(blk * 128, 128)],
                                    outs[k].at[pl.ds(blk * 128, 128)])
            plsc.subcore_barrier()


def _sc_matvec(idx_pairs, const, chunks, deg_mode=False):
    nchunks = 2 if deg_mode else len(chunks)
    mesh = plsc.VectorSubcoreMesh(core_axis_name="c", subcore_axis_name="s")
    fn = pl.kernel(
        functools.partial(_matvec_body, nchunks, deg_mode),
        mesh=mesh,
        out_type=[jax.ShapeDtypeStruct((NP, 128), jnp.float32)] * nchunks,
        scratch_types=[
            pltpu.VMEM((2, 128), jnp.int32),
            pltpu.VMEM((2, 128), jnp.int32),
            pltpu.VMEM((2, 128), jnp.int32),
            pltpu.VMEM((2, 128), jnp.int32),
            pltpu.VMEM((128, 128), jnp.float32),
            pltpu.VMEM((128, 128), jnp.float32),
            pltpu.VMEM_SHARED((NP, 128), jnp.float32),
            pltpu.SemaphoreType.DMA,
            pltpu.SemaphoreType.DMA,
            pltpu.SemaphoreType.DMA,
            pltpu.SemaphoreType.DMA,
            pltpu.SemaphoreType.DMA,
            pltpu.SemaphoreType.DMA,
            pltpu.SemaphoreType.DMA,
            pltpu.SemaphoreType.DMA,
        ],
        compiler_params=pltpu.CompilerParams(needs_layout_passes=False),
    )
    return fn(idx_pairs, const, *chunks)


def _scatter_sum(idx_pairs, zeros, vs):
    f = vs.shape[1]
    chunks = [lax.slice_in_dim(vs, 128 * i, 128 * (i + 1), axis=1)
              for i in range(f // 128)]
    outs = _sc_matvec(idx_pairs, zeros, chunks)
    return jnp.concatenate(list(outs), axis=1)


def _remap_body(r_ref, c_ref, rp_ref, rd_ref):
    r = r_ref[...]
    c = c_ref[...]
    sm = r == c
    rp_ref[...] = jnp.where(sm, N, r)
    rd_ref[...] = jnp.where(sm, N + 8, r)


def _remap(row2d, col2d):
    nb = EP // (128 * 128)
    return pl.pallas_call(
        _remap_body,
        grid=(nb,),
        in_specs=[pl.BlockSpec((128, 128), lambda i: (i, 0))] * 2,
        out_specs=[pl.BlockSpec((128, 128), lambda i: (i, 0))] * 2,
        out_shape=[jax.ShapeDtypeStruct((EP // 128, 128), jnp.int32)] * 2,
    )(row2d, col2d)


def _dis_body(deg_ref, dis_ref):
    d = deg_ref[:, 0:1]
    dis_ref[...] = jnp.where(d > 0, lax.rsqrt(d), 0.0)


def _reduce_dis(deg_s):
    return pl.pallas_call(
        _dis_body,
        grid=(NBLK,),
        in_specs=[pl.BlockSpec((128, 128), lambda i: (i, 0))],
        out_specs=pl.BlockSpec((128, 1), lambda i: (i, 0)),
        out_shape=jax.ShapeDtypeStruct((NP, 1), jnp.float32),
    )(deg_s)


def _ew_call(body, ins, n_out, f):
    blk = lambda i, j: (i, j)
    dspec = pl.BlockSpec((128, 1), lambda i, j: (i, 0))
    specs = [pl.BlockSpec((128, 128), blk) for _ in ins[:-1]] + [dspec]
    return pl.pallas_call(
        body,
        grid=(NBLK, f // 128),
        in_specs=specs,
        out_specs=[pl.BlockSpec((128, 128), blk) for _ in range(n_out)],
        out_shape=[jax.ShapeDtypeStruct((NP, f), jnp.float32)] * n_out,
    )(*ins)


def _scale_body(x, dis, o):
    o[...] = dis[...] * x[...]


def _t1_body(x, s1, dis, t1, t1s):
    d = dis[...]
    t = x[...] - d * s1[...]
    t1[...] = t
    t1s[...] = d * t


def _t2_body(t1, s2, dis, t2):
    t2[...] = t1[...] - dis[...] * s2[...]


def _s_body(u1, u2, s3, dis, s_o, ss_o):
    d = dis[...]
    s = u1[...] + 2.0 * u2[...] - 2.0 * d * s3[...]
    s_o[...] = s
    ss_o[...] = d * s


def _out_body(a, s, s4, dis, o):
    o[...] = a[...] + s[...] - dis[...] * s4[...]


def _mm1_body(x0, x1, x2, w0, w1, w2, b_ref, o_ref):
    acc = jnp.dot(x0[...].astype(jnp.bfloat16), w0[...],
                  preferred_element_type=jnp.float32)
    acc += jnp.dot(x1[...].astype(jnp.bfloat16), w1[...],
                   preferred_element_type=jnp.float32)
    acc += jnp.dot(x2[...].astype(jnp.bfloat16), w2[...],
                   preferred_element_type=jnp.float32)
    o_ref[...] = jnp.maximum(acc + b_ref[...], 0.0)


def _mm1(x0, x1, x2, w0, w1, w2, b, rb=1264, cb=512):
    n = w0.shape[1]
    xspec = pl.BlockSpec((rb, 256), lambda i, j: (i, 0))
    wspec = pl.BlockSpec((256, cb), lambda i, j: (0, j))
    return pl.pallas_call(
        _mm1_body,
        grid=(NP // rb, n // cb),
        in_specs=[xspec, xspec, xspec, wspec, wspec, wspec,
                  pl.BlockSpec((1, cb), lambda i, j: (0, j))],
        out_specs=pl.BlockSpec((rb, cb), lambda i, j: (i, j)),
        out_shape=jax.ShapeDtypeStruct((NP, n), jnp.float32),
        compiler_params=pltpu.CompilerParams(
            dimension_semantics=("parallel", "parallel")),
    )(x0, x1, x2, w0, w1, w2, b)


def _u2_body(nk, x_ref, w_ref, dis_ref, u2_ref, u2s_ref):
    k = pl.program_id(2)

    @pl.when(k == 0)
    def _():
        u2_ref[...] = jnp.zeros_like(u2_ref)

    u2_ref[...] += jnp.dot(x_ref[...].astype(jnp.bfloat16), w_ref[...],
                           preferred_element_type=jnp.float32)

    @pl.when(k == nk - 1)
    def _():
        u2s_ref[...] = dis_ref[...] * u2_ref[...]


def _mm_u2(x, w, dis, rb=1264, cb=512, kb=512):
    m, kk = x.shape
    _, n = w.shape
    nk = kk // kb
    return pl.pallas_call(
        functools.partial(_u2_body, nk),
        grid=(m // rb, n // cb, nk),
        in_specs=[
            pl.BlockSpec((rb, kb), lambda i, j, k: (i, k)),
            pl.BlockSpec((kb, cb), lambda i, j, k: (k, j)),
            pl.BlockSpec((rb, 1), lambda i, j, k: (i, 0)),
        ],
        out_specs=[pl.BlockSpec((rb, cb), lambda i, j, k: (i, j))] * 2,
        out_shape=[jax.ShapeDtypeStruct((m, n), jnp.float32)] * 2,
        compiler_params=pltpu.CompilerParams(
            dimension_semantics=("parallel", "parallel", "arbitrary")),
    )(x, w, dis)


def _mm_body(nk, x_ref, w_ref, b_ref, o_ref):
    k = pl.program_id(2)

    @pl.when(k == 0)
    def _():
        o_ref[...] = jnp.zeros_like(o_ref)

    o_ref[...] += jnp.dot(x_ref[...].astype(jnp.bfloat16), w_ref[...],
                          preferred_element_type=jnp.float32)

    @pl.when(k == nk - 1)
    def _():
        o_ref[...] += b_ref[...]


def _mm(x, w, b, rb=1264, cb=512, kb=512):
    m, kk = x.shape
    _, n = w.shape
    nk = kk // kb
    return pl.pallas_call(
        functools.partial(_mm_body, nk),
        grid=(m // rb, n // cb, nk),
        in_specs=[
            pl.BlockSpec((rb, kb), lambda i, j, k: (i, k)),
            pl.BlockSpec((kb, cb), lambda i, j, k: (k, j)),
            pl.BlockSpec((1, cb), lambda i, j, k: (0, j)),
        ],
        out_specs=pl.BlockSpec((rb, cb), lambda i, j, k: (i, j)),
        out_shape=jax.ShapeDtypeStruct((m, n), jnp.float32),
        compiler_params=pltpu.CompilerParams(
            dimension_semantics=("parallel", "parallel", "arbitrary")),
    )(x, w, b)


def _band(w, in_ch, out_ch):
    li = jnp.arange(128)[:, None]
    lo = jnp.arange(128)[None, :]
    kk = li - lo + 8
    valid = (kk >= 0) & (kk <= 8)
    bm = w[:, :, jnp.clip(kk, 0, 8)]
    bm = jnp.where(valid[None, None], bm, 0.0)
    return bm.transpose(1, 2, 0, 3).reshape(in_ch * 128, out_ch * 128)


def kernel(x, edge_index, W1_0, W1_1, W1_2, b1_0, b1_1, b1_2,
           W2_0, W2_1, W2_2, b2_0, b2_1, b2_2, bias_1, bias_2):
    row = jnp.concatenate([edge_index[0], jnp.full((EP - E,), N, jnp.int32)])
    col = jnp.concatenate([edge_index[1], jnp.full((EP - E,), N, jnp.int32)])

    M0 = _band(W1_0, 2, 16)
    M1 = _band(W1_1, 2, 16)
    M2 = _band(W1_2, 2, 16)
    N0 = _band(W2_0, 16, 8)
    N1 = _band(W2_1, 16, 8)
    N2 = _band(W2_2, 16, 8)
    Nab = jnp.concatenate([N0 - N2, N1], axis=1)
    b1f = jnp.repeat(b1_0 + b1_1 + b1_2, 128)[None, :] + bias_1
    b2f = jnp.repeat(b2_0 + b2_1 + b2_2, 128)[None, :] + bias_2
    bab = jnp.concatenate([b2f, jnp.zeros((1, 1024), jnp.float32)], axis=1)

    xpad = jnp.concatenate([x, jnp.zeros((NP - N, 256), jnp.float32)])
    const = jnp.stack([jnp.zeros((128, 128), jnp.float32),
                       jnp.ones((128, 128), jnp.float32)])

    rowp2d, rdeg2d = _remap(row.reshape(EP // 128, 128),
                            col.reshape(EP // 128, 128))
    ix_main = jnp.stack([rowp2d.reshape(NS, NB, 128),
                         col.reshape(NS, NB, 128)], axis=2)
    rdeg3 = rdeg2d.reshape(NS, NB, 128)
    ix_deg = jnp.stack([rdeg3, rdeg3], axis=2)

    deg_s = _sc_matvec(ix_deg, const, [], deg_mode=True)[0]
    dis = _reduce_dis(deg_s)

    xs = _ew_call(_scale_body, [xpad, dis], 1, 256)[0]
    S1 = _scatter_sum(ix_main, const, xs)
    t1, t1s = _ew_call(_t1_body, [xpad, S1, dis], 2, 256)
    S2 = _scatter_sum(ix_main, const, t1s)
    t2 = _ew_call(_t2_body, [t1, S2, dis], 1, 256)[0]
    bf = jnp.bfloat16
    h = _mm1(xpad, t1, t2, (M0 - M2).astype(bf), M1.astype(bf),
             (2.0 * M2).astype(bf), b1f)

    u2, u2s = _mm_u2(h, N2.astype(bf), dis)
    S3 = _scatter_sum(ix_main, const, u2s)
    au1 = _mm(h, Nab.astype(bf), bab)
    a = lax.slice_in_dim(au1, 0, 1024, axis=1)
    u1 = lax.slice_in_dim(au1, 1024, 2048, axis=1)
    s, ss = _ew_call(_s_body, [u1, u2, S3, dis], 2, 1024)
    S4 = _scatter_sum(ix_main, const, ss)
    out = _ew_call(_out_body, [a, s, S4, dis], 1, 1024)[0]
    return out[:N]


# speedup vs baseline: 3.0868x; 1.0053x over previous
import functools

import jax
import jax.numpy as jnp
from jax import lax
from jax.experimental import pallas as pl
from jax.experimental.pallas import tpu as pltpu
from jax.experimental.pallas import tpu_sc as plsc

N = 10000
NP = 10112
E = 160000
EP = 163840
NC, NS = 2, 16
NW = NC * NS
EW = EP // NS
NB = EW // 128
NBLK = NP // 128


def _matvec_body(nchunks, deg_mode, *refs):
    nv = 0 if deg_mode else nchunks
    (ix_h, const_h), vs = refs[0:2], refs[2:2 + nv]
    outs = refs[2 + nv:2 + nv + nchunks]
    rest = refs[2 + nv + nchunks:]
    ibs = rest[0:4]
    rowss = rest[4:6]
    acc = rest[6]
    isems = rest[7:11]
    gsems = rest[11:13]
    ssems = rest[13:15]
    cid = lax.axis_index("c")
    sid = lax.axis_index("s")

    def idx_fire(b, q):
        pltpu.async_copy(ix_h.at[sid, b], ibs[q], isems[q])

    def idx_wait(b, q):
        pltpu.make_async_copy(ix_h.at[sid, b], ibs[q], isems[q]).wait()

    def g_fire(k, p, q):
        pltpu.async_copy(vs[k].at[ibs[q].at[0]], rowss[p], gsems[p])

    def g_wait(k, p, q):
        pltpu.make_async_copy(vs[k].at[ibs[q].at[0]], rowss[p],
                              gsems[p]).wait()

    def s_fire(p, q):
        pltpu.async_copy(rowss[p], acc.at[ibs[q].at[1]], ssems[p], add=True)

    def s_wait(p, q):
        pltpu.make_async_copy(rowss[p], acc.at[ibs[q].at[1]],
                              ssems[p]).wait()

    for k in range(nchunks):
        @pl.when(cid == k % NC)
        def _chunk(k=k):
            for j in range(5):
                blk = sid + j * NS
                @pl.when(blk < NBLK)
                def _z(blk=blk):
                    pltpu.sync_copy(const_h.at[0], acc.at[pl.ds(blk * 128, 128)])
            plsc.subcore_barrier()

            if deg_mode:
                pltpu.sync_copy(const_h.at[1], rowss[0])
                idx_fire(0, 0)
                idx_fire(1, 1)

                def quad(i, _):
                    for pos in range(4):
                        b = 4 * i + pos
                        p, q = pos % 2, pos
                        @pl.when(b >= 2)
                        def _(p=p, q=q):
                            s_wait(p, (q + 2) % 4)
                        @pl.when(b + 2 < NB)
                        def _(b=b, q=q):
                            idx_fire(b + 2, (q + 2) % 4)
                        idx_wait(b, q)
                        pltpu.async_copy(rowss[0], acc.at[ibs[q].at[1]],
                                         ssems[p], add=True)
                    return _
                lax.fori_loop(0, NB // 4, quad, None)
                s_wait(0, 2)
                s_wait(1, 3)
            else:
                idx_fire(0, 0)
                idx_fire(1, 1)
                idx_fire(2, 2)
                idx_wait(0, 0)
                g_fire(k, 0, 0)

                def quad(i, _):
                    for pos in range(4):
                        b = 4 * i + pos
                        p, q = pos % 2, pos
                        pn, qn = (pos + 1) % 2, (pos + 1) % 4
                        @pl.when(b >= 1)
                        def _(pn=pn, q=q):
                            s_wait(pn, (q + 3) % 4)
                        @pl.when(b + 3 < NB)
                        def _(b=b, q=q):
                            idx_fire(b + 3, (q + 3) % 4)
                        @pl.when(b + 1 < NB)
                        def _(b=b, pn=pn, qn=qn, k=k):
                            idx_wait(b + 1, qn)
                            g_fire(k, pn, qn)
                        g_wait(k, p, q)
                        s_fire(p, q)
                    return _
                lax.fori_loop(0, NB // 4, quad, None)
                s_wait(1, 3)
            plsc.subcore_barrier()

            for j in range(5):
                blk = sid + j * NS
                @pl.when(blk < NBLK)
                def _f(blk=blk, k=k):
                    pltpu.sync_copy(acc.at[pl.ds(blk * 128, 128)],
                                    outs[k].at[pl.ds(blk * 128, 128)])
            plsc.subcore_barrier()


def _sc_matvec(idx_pairs, const, chunks, deg_mode=False):
    nchunks = 2 if deg_mode else len(chunks)
    mesh = plsc.VectorSubcoreMesh(core_axis_name="c", subcore_axis_name="s")
    fn = pl.kernel(
        functools.partial(_matvec_body, nchunks, deg_mode),
        mesh=mesh,
        out_type=[jax.ShapeDtypeStruct((NP, 128), jnp.float32)] * nchunks,
        scratch_types=[
            pltpu.VMEM((2, 128), jnp.int32),
            pltpu.VMEM((2, 128), jnp.int32),
            pltpu.VMEM((2, 128), jnp.int32),
            pltpu.VMEM((2, 128), jnp.int32),
            pltpu.VMEM((128, 128), jnp.float32),
            pltpu.VMEM((128, 128), jnp.float32),
            pltpu.VMEM_SHARED((NP, 128), jnp.float32),
            pltpu.SemaphoreType.DMA,
            pltpu.SemaphoreType.DMA,
            pltpu.SemaphoreType.DMA,
            pltpu.SemaphoreType.DMA,
            pltpu.SemaphoreType.DMA,
            pltpu.SemaphoreType.DMA,
            pltpu.SemaphoreType.DMA,
            pltpu.SemaphoreType.DMA,
        ],
        compiler_params=pltpu.CompilerParams(needs_layout_passes=False),
    )
    return fn(idx_pairs, const, *chunks)


def _scatter_sum(idx_pairs, zeros, vs):
    f = vs.shape[1]
    chunks = [lax.slice_in_dim(vs, 128 * i, 128 * (i + 1), axis=1)
              for i in range(f // 128)]
    outs = _sc_matvec(idx_pairs, zeros, chunks)
    return jnp.concatenate(list(outs), axis=1)


def _remap_body(r_ref, c_ref, rp_ref, rd_ref):
    r = r_ref[...]
    c = c_ref[...]
    sm = r == c
    rp_ref[...] = jnp.where(sm, N, r)
    rd_ref[...] = jnp.where(sm, N + 8, r)


def _remap(row2d, col2d):
    nb = EP // (128 * 128)
    return pl.pallas_call(
        _remap_body,
        grid=(nb,),
        in_specs=[pl.BlockSpec((128, 128), lambda i: (i, 0))] * 2,
        out_specs=[pl.BlockSpec((128, 128), lambda i: (i, 0))] * 2,
        out_shape=[jax.ShapeDtypeStruct((EP // 128, 128), jnp.int32)] * 2,
    )(row2d, col2d)


def _dis_body(deg_ref, dis_ref):
    d = deg_ref[:, 0:1]
    dis_ref[...] = jnp.where(d > 0, lax.rsqrt(d), 0.0)


def _reduce_dis(deg_s):
    return pl.pallas_call(
        _dis_body,
        grid=(NBLK,),
        in_specs=[pl.BlockSpec((128, 128), lambda i: (i, 0))],
        out_specs=pl.BlockSpec((128, 1), lambda i: (i, 0)),
        out_shape=jax.ShapeDtypeStruct((NP, 1), jnp.float32),
    )(deg_s)


def _ew_call(body, ins, n_out, f):
    blk = lambda i, j: (i, j)
    dspec = pl.BlockSpec((128, 1), lambda i, j: (i, 0))
    specs = [pl.BlockSpec((128, 128), blk) for _ in ins[:-1]] + [dspec]
    return pl.pallas_call(
        body,
        grid=(NBLK, f // 128),
        in_specs=specs,
        out_specs=[pl.BlockSpec((128, 128), blk) for _ in range(n_out)],
        out_shape=[jax.ShapeDtypeStruct((NP, f), jnp.float32)] * n_out,
    )(*ins)


def _scale_body(x, dis, o):
    o[...] = dis[...] * x[...]


def _t1_body(x, s1, dis, t1, t1s):
    d = dis[...]
    t = x[...] - d * s1[...]
    t1[...] = t
    t1s[...] = d * t


def _t2_body(t1, s2, dis, t2):
    t2[...] = t1[...] - dis[...] * s2[...]


def _s_body(u1, u2, s3, dis, s_o, ss_o):
    d = dis[...]
    s = u1[...] + 2.0 * u2[...] - 2.0 * d * s3[...]
    s_o[...] = s
    ss_o[...] = d * s


def _out_body(a, s, s4, dis, o):
    o[...] = a[...] + s[...] - dis[...] * s4[...]


def _acc_body(relu, x_ref, w_ref, a_ref, b_ref, o_ref):
    acc = a_ref[...] + jnp.dot(x_ref[...].astype(jnp.bfloat16), w_ref[...],
                               preferred_element_type=jnp.float32)
    if relu:
        acc = jnp.maximum(acc + b_ref[...], 0.0)
    o_ref[...] = acc


def _mm_acc(x, w, a, b=None, rb=1264, cb=512):
    n = w.shape[1]
    relu = b is not None
    ins = [x, w, a] + ([b] if relu else [])
    specs = [pl.BlockSpec((rb, 256), lambda i, j: (i, 0)),
             pl.BlockSpec((256, cb), lambda i, j: (0, j)),
             pl.BlockSpec((rb, cb), lambda i, j: (i, j))]
    if relu:
        specs.append(pl.BlockSpec((1, cb), lambda i, j: (0, j)))
    body = (functools.partial(_acc_body, True) if relu else
            lambda x_, w_, a_, o_: _acc_body(False, x_, w_, a_, None, o_))
    return pl.pallas_call(
        body,
        grid=(NP // rb, n // cb),
        in_specs=specs,
        out_specs=pl.BlockSpec((rb, cb), lambda i, j: (i, j)),
        out_shape=jax.ShapeDtypeStruct((NP, n), jnp.float32),
        compiler_params=pltpu.CompilerParams(
            dimension_semantics=("parallel", "parallel")),
    )(*ins)


def _u2_body(nk, x_ref, w_ref, dis_ref, u2_ref, u2s_ref):
    k = pl.program_id(2)

    @pl.when(k == 0)
    def _():
        u2_ref[...] = jnp.zeros_like(u2_ref)

    u2_ref[...] += jnp.dot(x_ref[...].astype(jnp.bfloat16), w_ref[...],
                           preferred_element_type=jnp.float32)

    @pl.when(k == nk - 1)
    def _():
        u2s_ref[...] = dis_ref[...] * u2_ref[...]


def _mm_u2(x, w, dis, rb=1264, cb=512, kb=512):
    m, kk = x.shape
    _, n = w.shape
    nk = kk // kb
    return pl.pallas_call(
        functools.partial(_u2_body, nk),
        grid=(m // rb, n // cb, nk),
        in_specs=[
            pl.BlockSpec((rb, kb), lambda i, j, k: (i, k)),
            pl.BlockSpec((kb, cb), lambda i, j, k: (k, j)),
            pl.BlockSpec((rb, 1), lambda i, j, k: (i, 0)),
        ],
        out_specs=[pl.BlockSpec((rb, cb), lambda i, j, k: (i, j))] * 2,
        out_shape=[jax.ShapeDtypeStruct((m, n), jnp.float32)] * 2,
        compiler_params=pltpu.CompilerParams(
            dimension_semantics=("parallel", "parallel", "arbitrary")),
    )(x, w, dis)


def _mm_body(nk, x_ref, w_ref, b_ref, o_ref):
    k = pl.program_id(2)

    @pl.when(k == 0)
    def _():
        o_ref[...] = jnp.zeros_like(o_ref)

    o_ref[...] += jnp.dot(x_ref[...].astype(jnp.bfloat16), w_ref[...],
                          preferred_element_type=jnp.float32)

    @pl.when(k == nk - 1)
    def _():
        o_ref[...] += b_ref[...]


def _mm(x, w, b, rb=1264, cb=512, kb=512):
    m, kk = x.shape
    _, n = w.shape
    nk = kk // kb
    return pl.pallas_call(
        functools.partial(_mm_body, nk),
        grid=(m // rb, n // cb, nk),
        in_specs=[
            pl.BlockSpec((rb, kb), lambda i, j, k: (i, k)),
            pl.BlockSpec((kb, cb), lambda i, j, k: (k, j)),
            pl.BlockSpec((1, cb), lambda i, j, k: (0, j)),
        ],
        out_specs=pl.BlockSpec((rb, cb), lambda i, j, k: (i, j)),
        out_shape=jax.ShapeDtypeStruct((m, n), jnp.float32),
        compiler_params=pltpu.CompilerParams(
            dimension_semantics=("parallel", "parallel", "arbitrary")),
    )(x, w, b)


def _band(w, in_ch, out_ch):
    li = jnp.arange(128)[:, None]
    lo = jnp.arange(128)[None, :]
    kk = li - lo + 8
    valid = (kk >= 0) & (kk <= 8)
    bm = w[:, :, jnp.clip(kk, 0, 8)]
    bm = jnp.where(valid[None, None], bm, 0.0)
    return bm.transpose(1, 2, 0, 3).reshape(in_ch * 128, out_ch * 128)


def kernel(x, edge_index, W1_0, W1_1, W1_2, b1_0, b1_1, b1_2,
           W2_0, W2_1, W2_2, b2_0, b2_1, b2_2, bias_1, bias_2):
    row = jnp.concatenate([edge_index[0], jnp.full((EP - E,), N, jnp.int32)])
    col = jnp.concatenate([edge_index[1], jnp.full((EP - E,), N, jnp.int32)])

    M0 = _band(W1_0, 2, 16)
    M1 = _band(W1_1, 2, 16)
    M2 = _band(W1_2, 2, 16)
    N0 = _band(W2_0, 16, 8)
    N1 = _band(W2_1, 16, 8)
    N2 = _band(W2_2, 16, 8)
    b1f = jnp.repeat(b1_0 + b1_1 + b1_2, 128)[None, :] + bias_1
    b2f = jnp.repeat(b2_0 + b2_1 + b2_2, 128)[None, :] + bias_2

    xpad = jnp.concatenate([x, jnp.zeros((NP - N, 256), jnp.float32)])
    const = jnp.stack([jnp.zeros((128, 128), jnp.float32),
                       jnp.ones((128, 128), jnp.float32)])

    rowp2d, rdeg2d = _remap(row.reshape(EP // 128, 128),
                            col.reshape(EP // 128, 128))
    ix_main = jnp.stack([rowp2d.reshape(NS, NB, 128),
                         col.reshape(NS, NB, 128)], axis=2)
    rdeg3 = rdeg2d.reshape(NS, NB, 128)
    ix_deg = jnp.stack([rdeg3, rdeg3], axis=2)

    deg_s = _sc_matvec(ix_deg, const, [], deg_mode=True)[0]
    dis = _reduce_dis(deg_s)

    bf = jnp.bfloat16
    xs = _ew_call(_scale_body, [xpad, dis], 1, 256)[0]
    S1 = _scatter_sum(ix_main, const, xs)
    hA = _mm(xpad, (M0 - M2).astype(bf),
             jnp.zeros((1, 2048), jnp.float32), kb=256)
    t1, t1s = _ew_call(_t1_body, [xpad, S1, dis], 2, 256)
    S2 = _scatter_sum(ix_main, const, t1s)
    hB = _mm_acc(t1, M1.astype(bf), hA)
    t2 = _ew_call(_t2_body, [t1, S2, dis], 1, 256)[0]
    h = _mm_acc(t2, (2.0 * M2).astype(bf), hB, b1f)

    u2, u2s = _mm_u2(h, N2.astype(bf), dis)
    S3 = _scatter_sum(ix_main, const, u2s)
    u1 = _mm(h, N1.astype(bf), jnp.zeros((1, 1024), jnp.float32))
    s, ss = _ew_call(_s_body, [u1, u2, S3, dis], 2, 1024)
    S4 = _scatter_sum(ix_main, const, ss)
    a = _mm(h, (N0 - N2).astype(bf), b2f)
    out = _ew_call(_out_body, [a, s, S4, dis], 1, 1024)[0]
    return out[:N]


# speedup vs baseline: 3.6932x; 1.1964x over previous
import functools

import jax
import jax.numpy as jnp
from jax import lax
from jax.experimental import pallas as pl
from jax.experimental.pallas import tpu as pltpu
from jax.experimental.pallas import tpu_sc as plsc

N = 10000
NP = 10112
E = 160000
EP = 163840
NC, NS = 2, 16
NW = NC * NS
EW = EP // NS
NB = EW // 128
NBLK = NP // 128


def _matvec_body(nchunks, deg_mode, *refs):
    nv = 0 if deg_mode else nchunks
    (ix_h, const_h), vs = refs[0:2], refs[2:2 + nv]
    outs = refs[2 + nv:2 + nv + nchunks]
    rest = refs[2 + nv + nchunks:]
    ibs = rest[0:4]
    rowss = rest[4:6]
    acc = rest[6]
    isems = rest[7:11]
    gsems = rest[11:13]
    ssems = rest[13:15]
    cid = lax.axis_index("c")
    sid = lax.axis_index("s")

    def idx_fire(b, q):
        pltpu.async_copy(ix_h.at[sid, b], ibs[q], isems[q])

    def idx_wait(b, q):
        pltpu.make_async_copy(ix_h.at[sid, b], ibs[q], isems[q]).wait()

    def g_fire(k, p, q):
        pltpu.async_copy(vs[k].at[ibs[q].at[0]], rowss[p], gsems[p])

    def g_wait(k, p, q):
        pltpu.make_async_copy(vs[k].at[ibs[q].at[0]], rowss[p],
                              gsems[p]).wait()

    def s_fire(p, q):
        pltpu.async_copy(rowss[p], acc.at[ibs[q].at[1]], ssems[p], add=True)

    def s_wait(p, q):
        pltpu.make_async_copy(rowss[p], acc.at[ibs[q].at[1]],
                              ssems[p]).wait()

    for k in range(nchunks):
        @pl.when(cid == k % NC)
        def _chunk(k=k):
            for j in range(5):
                blk = sid + j * NS
                @pl.when(blk < NBLK)
                def _z(blk=blk):
                    pltpu.sync_copy(const_h.at[0], acc.at[pl.ds(blk * 128, 128)])
            plsc.subcore_barrier()

            if deg_mode:
                pltpu.sync_copy(const_h.at[1], rowss[0])
                idx_fire(0, 0)
                idx_fire(1, 1)

                def quad(i, _):
                    for pos in range(4):
                        b = 4 * i + pos
                        p, q = pos % 2, pos
                        @pl.when(b >= 2)
                        def _(p=p, q=q):
                            s_wait(p, (q + 2) % 4)
                        @pl.when(b + 2 < NB)
                        def _(b=b, q=q):
                            idx_fire(b + 2, (q + 2) % 4)
                        idx_wait(b, q)
                        pltpu.async_copy(rowss[0], acc.at[ibs[q].at[1]],
                                         ssems[p], add=True)
                    return _
                lax.fori_loop(0, NB // 4, quad, None)
                s_wait(0, 2)
                s_wait(1, 3)
            else:
                idx_fire(0, 0)
                idx_fire(1, 1)
                idx_fire(2, 2)
                idx_wait(0, 0)
                g_fire(k, 0, 0)

                def quad(i, _):
                    for pos in range(4):
                        b = 4 * i + pos
                        p, q = pos % 2, pos
                        pn, qn = (pos + 1) % 2, (pos + 1) % 4
                        @pl.when(b >= 1)
                        def _(pn=pn, q=q):
                            s_wait(pn, (q + 3) % 4)
                        @pl.when(b + 3 < NB)
                        def _(b=b, q=q):
                            idx_fire(b + 3, (q + 3) % 4)
                        @pl.when(b + 1 < NB)
                        def _(b=b, pn=pn, qn=qn, k=k):
                            idx_wait(b + 1, qn)
                            g_fire(k, pn, qn)
                        g_wait(k, p, q)
                        s_fire(p, q)
                    return _
                lax.fori_loop(0, NB // 4, quad, None)
                s_wait(1, 3)
            plsc.subcore_barrier()

            for j in range(5):
                blk = sid + j * NS
                @pl.when(blk < NBLK)
                def _f(blk=blk, k=k):
                    pltpu.sync_copy(acc.at[pl.ds(blk * 128, 128)],
                                    outs[k].at[pl.ds(blk * 128, 128)])
            plsc.subcore_barrier()


def _sc_matvec(idx_pairs, const, chunks, deg_mode=False):
    nchunks = 2 if deg_mode else len(chunks)
    mesh = plsc.VectorSubcoreMesh(core_axis_name="c", subcore_axis_name="s")
    fn = pl.kernel(
        functools.partial(_matvec_body, nchunks, deg_mode),
        mesh=mesh,
        out_type=[jax.ShapeDtypeStruct((NP, 128), jnp.float32)] * nchunks,
        scratch_types=[
            pltpu.VMEM((2, 128), jnp.int32),
            pltpu.VMEM((2, 128), jnp.int32),
            pltpu.VMEM((2, 128), jnp.int32),
            pltpu.VMEM((2, 128), jnp.int32),
            pltpu.VMEM((128, 128), jnp.float32),
            pltpu.VMEM((128, 128), jnp.float32),
            pltpu.VMEM_SHARED((NP, 128), jnp.float32),
            pltpu.SemaphoreType.DMA,
            pltpu.SemaphoreType.DMA,
            pltpu.SemaphoreType.DMA,
            pltpu.SemaphoreType.DMA,
            pltpu.SemaphoreType.DMA,
            pltpu.SemaphoreType.DMA,
            pltpu.SemaphoreType.DMA,
            pltpu.SemaphoreType.DMA,
        ],
        compiler_params=pltpu.CompilerParams(needs_layout_passes=False),
    )
    return fn(idx_pairs, const, *chunks)


def _scatter_sum(idx_pairs, zeros, vs):
    f = vs.shape[1]
    chunks = [lax.slice_in_dim(vs, 128 * i, 128 * (i + 1), axis=1)
              for i in range(f // 128)]
    outs = _sc_matvec(idx_pairs, zeros, chunks)
    return jnp.concatenate(list(outs), axis=1)


def _remap_body(r_ref, c_ref, rp_ref, rd_ref):
    r = r_ref[...]
    c = c_ref[...]
    sm = r == c
    rp_ref[...] = jnp.where(sm, N, r)
    rd_ref[...] = jnp.where(sm, N + 8, r)


def _remap(row2d, col2d):
    nb = EP // (128 * 128)
    return pl.pallas_call(
        _remap_body,
        grid=(nb,),
        in_specs=[pl.BlockSpec((128, 128), lambda i: (i, 0))] * 2,
        out_specs=[pl.BlockSpec((128, 128), lambda i: (i, 0))] * 2,
        out_shape=[jax.ShapeDtypeStruct((EP // 128, 128), jnp.int32)] * 2,
    )(row2d, col2d)


def _dis_body(deg_ref, dis_ref):
    d = deg_ref[:, 0:1]
    dis_ref[...] = jnp.where(d > 0, lax.rsqrt(d), 0.0)


def _reduce_dis(deg_s):
    return pl.pallas_call(
        _dis_body,
        grid=(NBLK,),
        in_specs=[pl.BlockSpec((128, 128), lambda i: (i, 0))],
        out_specs=pl.BlockSpec((128, 1), lambda i: (i, 0)),
        out_shape=jax.ShapeDtypeStruct((NP, 1), jnp.float32),
    )(deg_s)


def _ew_call(body, ins, n_out, f):
    rb, cb = 1264, min(f, 512)
    blk = lambda i, j: (i, j)
    dspec = pl.BlockSpec((rb, 1), lambda i, j: (i, 0))
    specs = [pl.BlockSpec((rb, cb), blk) for _ in ins[:-1]] + [dspec]
    return pl.pallas_call(
        body,
        grid=(NP // rb, f // cb),
        in_specs=specs,
        out_specs=[pl.BlockSpec((rb, cb), blk) for _ in range(n_out)],
        out_shape=[jax.ShapeDtypeStruct((NP, f), jnp.float32)] * n_out,
    )(*ins)


def _scale_body(x, dis, o):
    o[...] = dis[...] * x[...]


def _t1_body(x, s1, dis, t1, t1s):
    d = dis[...]
    t = x[...] - d * s1[...]
    t1[...] = t
    t1s[...] = d * t


def _t2_body(t1, s2, dis, t2):
    t2[...] = t1[...] - dis[...] * s2[...]


def _s_body(u1, u2, s3, dis, s_o, ss_o):
    d = dis[...]
    s = u1[...] + 2.0 * u2[...] - 2.0 * d * s3[...]
    s_o[...] = s
    ss_o[...] = d * s


def _out_body(a, s, s4, dis, o):
    o[...] = a[...] + s[...] - dis[...] * s4[...]


def _acc_body(relu, x_ref, w_ref, a_ref, b_ref, o_ref):
    acc = a_ref[...] + jnp.dot(x_ref[...].astype(jnp.bfloat16), w_ref[...],
                               preferred_element_type=jnp.float32)
    if relu:
        acc = jnp.maximum(acc + b_ref[...], 0.0)
    o_ref[...] = acc


def _mm_acc(x, w, a, b=None, rb=1264, cb=512):
    n = w.shape[1]
    relu = b is not None
    ins = [x, w, a] + ([b] if relu else [])
    specs = [pl.BlockSpec((rb, 256), lambda i, j: (i, 0)),
             pl.BlockSpec((256, cb), lambda i, j: (0, j)),
             pl.BlockSpec((rb, cb), lambda i, j: (i, j))]
    if relu:
        specs.append(pl.BlockSpec((1, cb), lambda i, j: (0, j)))
    body = (functools.partial(_acc_body, True) if relu else
            lambda x_, w_, a_, o_: _acc_body(False, x_, w_, a_, None, o_))
    return pl.pallas_call(
        body,
        grid=(NP // rb, n // cb),
        in_specs=specs,
        out_specs=pl.BlockSpec((rb, cb), lambda i, j: (i, j)),
        out_shape=jax.ShapeDtypeStruct((NP, n), jnp.float32),
        compiler_params=pltpu.CompilerParams(
            dimension_semantics=("parallel", "parallel")),
    )(*ins)


def _u2_body(nk, x_ref, w_ref, dis_ref, u2_ref, u2s_ref):
    k = pl.program_id(2)

    @pl.when(k == 0)
    def _():
        u2_ref[...] = jnp.zeros_like(u2_ref)

    u2_ref[...] += jnp.dot(x_ref[...].astype(jnp.bfloat16), w_ref[...],
                           preferred_element_type=jnp.float32)

    @pl.when(k == nk - 1)
    def _():
        u2s_ref[...] = dis_ref[...] * u2_ref[...]


def _mm_u2(x, w, dis, rb=1264, cb=512, kb=512):
    m, kk = x.shape
    _, n = w.shape
    nk = kk // kb
    return pl.pallas_call(
        functools.partial(_u2_body, nk),
        grid=(m // rb, n // cb, nk),
        in_specs=[
            pl.BlockSpec((rb, kb), lambda i, j, k: (i, k)),
            pl.BlockSpec((kb, cb), lambda i, j, k: (k, j)),
            pl.BlockSpec((rb, 1), lambda i, j, k: (i, 0)),
        ],
        out_specs=[pl.BlockSpec((rb, cb), lambda i, j, k: (i, j))] * 2,
        out_shape=[jax.ShapeDtypeStruct((m, n), jnp.float32)] * 2,
        compiler_params=pltpu.CompilerParams(
            dimension_semantics=("parallel", "parallel", "arbitrary")),
    )(x, w, dis)


def _mm_body(nk, x_ref, w_ref, b_ref, o_ref):
    k = pl.program_id(2)

    @pl.when(k == 0)
    def _():
        o_ref[...] = jnp.zeros_like(o_ref)

    o_ref[...] += jnp.dot(x_ref[...].astype(jnp.bfloat16), w_ref[...],
                          preferred_element_type=jnp.float32)

    @pl.when(k == nk - 1)
    def _():
        o_ref[...] += b_ref[...]


def _mm(x, w, b, rb=1264, cb=512, kb=512):
    m, kk = x.shape
    _, n = w.shape
    nk = kk // kb
    return pl.pallas_call(
        functools.partial(_mm_body, nk),
        grid=(m // rb, n // cb, nk),
        in_specs=[
            pl.BlockSpec((rb, kb), lambda i, j, k: (i, k)),
            pl.BlockSpec((kb, cb), lambda i, j, k: (k, j)),
            pl.BlockSpec((1, cb), lambda i, j, k: (0, j)),
        ],
        out_specs=pl.BlockSpec((rb, cb), lambda i, j, k: (i, j)),
        out_shape=jax.ShapeDtypeStruct((m, n), jnp.float32),
        compiler_params=pltpu.CompilerParams(
            dimension_semantics=("parallel", "parallel", "arbitrary")),
    )(x, w, b)


def _band(w, in_ch, out_ch):
    li = jnp.arange(128)[:, None]
    lo = jnp.arange(128)[None, :]
    kk = li - lo + 8
    valid = (kk >= 0) & (kk <= 8)
    bm = w[:, :, jnp.clip(kk, 0, 8)]
    bm = jnp.where(valid[None, None], bm, 0.0)
    return bm.transpose(1, 2, 0, 3).reshape(in_ch * 128, out_ch * 128)


def kernel(x, edge_index, W1_0, W1_1, W1_2, b1_0, b1_1, b1_2,
           W2_0, W2_1, W2_2, b2_0, b2_1, b2_2, bias_1, bias_2):
    row = jnp.concatenate([edge_index[0], jnp.full((EP - E,), N, jnp.int32)])
    col = jnp.concatenate([edge_index[1], jnp.full((EP - E,), N, jnp.int32)])

    M0 = _band(W1_0, 2, 16)
    M1 = _band(W1_1, 2, 16)
    M2 = _band(W1_2, 2, 16)
    N0 = _band(W2_0, 16, 8)
    N1 = _band(W2_1, 16, 8)
    N2 = _band(W2_2, 16, 8)
    b1f = jnp.repeat(b1_0 + b1_1 + b1_2, 128)[None, :] + bias_1
    b2f = jnp.repeat(b2_0 + b2_1 + b2_2, 128)[None, :] + bias_2

    xpad = jnp.concatenate([x, jnp.zeros((NP - N, 256), jnp.float32)])
    const = jnp.stack([jnp.zeros((128, 128), jnp.float32),
                       jnp.ones((128, 128), jnp.float32)])

    rowp2d, rdeg2d = _remap(row.reshape(EP // 128, 128),
                            col.reshape(EP // 128, 128))
    ix_main = jnp.stack([rowp2d.reshape(NS, NB, 128),
                         col.reshape(NS, NB, 128)], axis=2)
    rdeg3 = rdeg2d.reshape(NS, NB, 128)
    ix_deg = jnp.stack([rdeg3, rdeg3], axis=2)

    deg_s = _sc_matvec(ix_deg, const, [], deg_mode=True)[0]
    dis = _reduce_dis(deg_s)

    bf = jnp.bfloat16
    xs = _ew_call(_scale_body, [xpad, dis], 1, 256)[0]
    S1 = _scatter_sum(ix_main, const, xs)
    hA = _mm(xpad, (M0 - M2).astype(bf),
             jnp.zeros((1, 2048), jnp.float32), kb=256)
    t1, t1s = _ew_call(_t1_body, [xpad, S1, dis], 2, 256)
    S2 = _scatter_sum(ix_main, const, t1s)
    hB = _mm_acc(t1, M1.astype(bf), hA)
    t2 = _ew_call(_t2_body, [t1, S2, dis], 1, 256)[0]
    h = _mm_acc(t2, (2.0 * M2).astype(bf), hB, b1f)

    u2, u2s = _mm_u2(h, N2.astype(bf), dis)
    S3 = _scatter_sum(ix_main, const, u2s)
    u1 = _mm(h, N1.astype(bf), jnp.zeros((1, 1024), jnp.float32))
    s, ss = _ew_call(_s_body, [u1, u2, S3, dis], 2, 1024)
    S4 = _scatter_sum(ix_main, const, ss)
    a = _mm(h, (N0 - N2).astype(bf), b2f)
    out = _ew_call(_out_body, [a, s, S4, dis], 1, 1024)[0]
    return out[:N]


# speedup vs baseline: 3.8227x; 1.0351x over previous
import functools

import jax
import jax.numpy as jnp
from jax import lax
from jax.experimental import pallas as pl
from jax.experimental.pallas import tpu as pltpu
from jax.experimental.pallas import tpu_sc as plsc

N = 10000
NP = 10112
E = 160000
EP = 163840
NC, NS = 2, 16
NW = NC * NS
EW = EP // NS
NB = EW // 128
NBLK = NP // 128


def _matvec_body(nchunks, deg_mode, *refs):
    nv = 0 if deg_mode else nchunks
    (ix_h, const_h), vs = refs[0:2], refs[2:2 + nv]
    outs = refs[2 + nv:2 + nv + nchunks]
    rest = refs[2 + nv + nchunks:]
    ibs = rest[0:4]
    rowss = rest[4:6]
    acc = rest[6]
    isems = rest[7:11]
    gsems = rest[11:13]
    ssems = rest[13:15]
    cid = lax.axis_index("c")
    sid = lax.axis_index("s")

    def idx_fire(b, q):
        pltpu.async_copy(ix_h.at[sid, b], ibs[q], isems[q])

    def idx_wait(b, q):
        pltpu.make_async_copy(ix_h.at[sid, b], ibs[q], isems[q]).wait()

    def g_fire(k, p, q):
        pltpu.async_copy(vs[k].at[ibs[q].at[0]], rowss[p], gsems[p])

    def g_wait(k, p, q):
        pltpu.make_async_copy(vs[k].at[ibs[q].at[0]], rowss[p],
                              gsems[p]).wait()

    def s_fire(p, q):
        pltpu.async_copy(rowss[p], acc.at[ibs[q].at[1]], ssems[p], add=True)

    def s_wait(p, q):
        pltpu.make_async_copy(rowss[p], acc.at[ibs[q].at[1]],
                              ssems[p]).wait()

    for k in range(nchunks):
        @pl.when(cid == k % NC)
        def _chunk(k=k):
            for j in range(5):
                blk = sid + j * NS
                @pl.when(blk < NBLK)
                def _z(blk=blk):
                    pltpu.sync_copy(const_h.at[0], acc.at[pl.ds(blk * 128, 128)])
            plsc.subcore_barrier()

            if deg_mode:
                pltpu.sync_copy(const_h.at[1], rowss[0])
                idx_fire(0, 0)
                idx_fire(1, 1)

                def quad(i, _):
                    for pos in range(4):
                        b = 4 * i + pos
                        p, q = pos % 2, pos
                        @pl.when(b >= 2)
                        def _(p=p, q=q):
                            s_wait(p, (q + 2) % 4)
                        @pl.when(b + 2 < NB)
                        def _(b=b, q=q):
                            idx_fire(b + 2, (q + 2) % 4)
                        idx_wait(b, q)
                        pltpu.async_copy(rowss[0], acc.at[ibs[q].at[1]],
                                         ssems[p], add=True)
                    return _
                lax.fori_loop(0, NB // 4, quad, None)
                s_wait(0, 2)
                s_wait(1, 3)
            else:
                idx_fire(0, 0)
                idx_fire(1, 1)
                idx_fire(2, 2)
                idx_wait(0, 0)
                g_fire(k, 0, 0)

                def quad(i, _):
                    for pos in range(4):
                        b = 4 * i + pos
                        p, q = pos % 2, pos
                        pn, qn = (pos + 1) % 2, (pos + 1) % 4
                        @pl.when(b >= 1)
                        def _(pn=pn, q=q):
                            s_wait(pn, (q + 3) % 4)
                        @pl.when(b + 3 < NB)
                        def _(b=b, q=q):
                            idx_fire(b + 3, (q + 3) % 4)
                        @pl.when(b + 1 < NB)
                        def _(b=b, pn=pn, qn=qn, k=k):
                            idx_wait(b + 1, qn)
                            g_fire(k, pn, qn)
                        g_wait(k, p, q)
                        s_fire(p, q)
                    return _
                lax.fori_loop(0, NB // 4, quad, None)
                s_wait(1, 3)
            plsc.subcore_barrier()

            for j in range(5):
                blk = sid + j * NS
                @pl.when(blk < NBLK)
                def _f(blk=blk, k=k):
                    pltpu.sync_copy(acc.at[pl.ds(blk * 128, 128)],
                                    outs[k].at[pl.ds(blk * 128, 128)])
            plsc.subcore_barrier()


def _sc_matvec(idx_pairs, const, chunks, deg_mode=False):
    nchunks = 2 if deg_mode else len(chunks)
    mesh = plsc.VectorSubcoreMesh(core_axis_name="c", subcore_axis_name="s")
    fn = pl.kernel(
        functools.partial(_matvec_body, nchunks, deg_mode),
        mesh=mesh,
        out_type=[jax.ShapeDtypeStruct((NP, 128), jnp.float32)] * nchunks,
        scratch_types=[
            pltpu.VMEM((2, 128), jnp.int32),
            pltpu.VMEM((2, 128), jnp.int32),
            pltpu.VMEM((2, 128), jnp.int32),
            pltpu.VMEM((2, 128), jnp.int32),
            pltpu.VMEM((128, 128), jnp.float32),
            pltpu.VMEM((128, 128), jnp.float32),
            pltpu.VMEM_SHARED((NP, 128), jnp.float32),
            pltpu.SemaphoreType.DMA,
            pltpu.SemaphoreType.DMA,
            pltpu.SemaphoreType.DMA,
            pltpu.SemaphoreType.DMA,
            pltpu.SemaphoreType.DMA,
            pltpu.SemaphoreType.DMA,
            pltpu.SemaphoreType.DMA,
            pltpu.SemaphoreType.DMA,
        ],
        compiler_params=pltpu.CompilerParams(needs_layout_passes=False),
    )
    return fn(idx_pairs, const, *chunks)


def _scatter_sum(idx_pairs, zeros, vs):
    f = vs.shape[1]
    chunks = [lax.slice_in_dim(vs, 128 * i, 128 * (i + 1), axis=1)
              for i in range(f // 128)]
    outs = _sc_matvec(idx_pairs, zeros, chunks)
    return jnp.concatenate(list(outs), axis=1)


def _remap_body(r_ref, c_ref, rp_ref, rd_ref):
    r = r_ref[...]
    c = c_ref[...]
    sm = r == c
    rp_ref[...] = jnp.where(sm, N, r)
    rd_ref[...] = jnp.where(sm, N + 8, r)


def _remap(row2d, col2d):
    nb = EP // (128 * 128)
    return pl.pallas_call(
        _remap_body,
        grid=(nb,),
        in_specs=[pl.BlockSpec((128, 128), lambda i: (i, 0))] * 2,
        out_specs=[pl.BlockSpec((128, 128), lambda i: (i, 0))] * 2,
        out_shape=[jax.ShapeDtypeStruct((EP // 128, 128), jnp.int32)] * 2,
    )(row2d, col2d)


def _dis_body(deg_ref, dis_ref):
    d = deg_ref[:, 0:1]
    dis_ref[...] = jnp.where(d > 0, lax.rsqrt(d), 0.0)


def _reduce_dis(deg_s):
    return pl.pallas_call(
        _dis_body,
        grid=(NBLK,),
        in_specs=[pl.BlockSpec((128, 128), lambda i: (i, 0))],
        out_specs=pl.BlockSpec((128, 1), lambda i: (i, 0)),
        out_shape=jax.ShapeDtypeStruct((NP, 1), jnp.float32),
    )(deg_s)


def _ew_call(body, ins, n_out, f):
    rb, cb = 1264, min(f, 512)
    blk = lambda i, j: (i, j)
    dspec = pl.BlockSpec((rb, 1), lambda i, j: (i, 0))
    specs = [pl.BlockSpec((rb, cb), blk) for _ in ins[:-1]] + [dspec]
    return pl.pallas_call(
        body,
        grid=(NP // rb, f // cb),
        in_specs=specs,
        out_specs=[pl.BlockSpec((rb, cb), blk) for _ in range(n_out)],
        out_shape=[jax.ShapeDtypeStruct((NP, f), jnp.float32)] * n_out,
    )(*ins)


def _scale_body(x, dis, o):
    o[...] = dis[...] * x[...]


def _t1_body(x, s1, dis, t1, t1s):
    d = dis[...]
    t = x[...] - d * s1[...]
    t1[...] = t
    t1s[...] = d * t


def _t2_body(t1, s2, dis, t2):
    t2[...] = t1[...] - dis[...] * s2[...]


def _s_body(u1, u2, s3, dis, s_o, ss_o):
    d = dis[...]
    s = u1[...] + 2.0 * u2[...] - 2.0 * d * s3[...]
    s_o[...] = s
    ss_o[...] = d * s


def _out_body(a, s, s4, dis, o):
    o[...] = a[...] + s[...] - dis[...] * s4[...]


def _acc_body(relu, x_ref, w_ref, a_ref, b_ref, o_ref):
    acc = a_ref[...] + jnp.dot(x_ref[...].astype(jnp.bfloat16), w_ref[...],
                               preferred_element_type=jnp.float32)
    if relu:
        acc = jnp.maximum(acc + b_ref[...], 0.0)
    o_ref[...] = acc


def _mm_acc(x, w, a, b=None, rb=1264, cb=512):
    n = w.shape[1]
    relu = b is not None
    ins = [x, w, a] + ([b] if relu else [])
    specs = [pl.BlockSpec((rb, 256), lambda i, j: (i, 0)),
             pl.BlockSpec((256, cb), lambda i, j: (0, j)),
             pl.BlockSpec((rb, cb), lambda i, j: (i, j))]
    if relu:
        specs.append(pl.BlockSpec((1, cb), lambda i, j: (0, j)))
    body = (functools.partial(_acc_body, True) if relu else
            lambda x_, w_, a_, o_: _acc_body(False, x_, w_, a_, None, o_))
    return pl.pallas_call(
        body,
        grid=(NP // rb, n // cb),
        in_specs=specs,
        out_specs=pl.BlockSpec((rb, cb), lambda i, j: (i, j)),
        out_shape=jax.ShapeDtypeStruct((NP, n), jnp.float32),
        compiler_params=pltpu.CompilerParams(
            dimension_semantics=("parallel", "parallel")),
    )(*ins)


def _l2head_body(t2_ref, hB_ref, w2_ref, b_ref, n2_ref, dis_ref,
                 h_ref, u2_ref, u2s_ref):
    h = hB_ref[...] + jnp.dot(t2_ref[...].astype(jnp.bfloat16), w2_ref[...],
                              preferred_element_type=jnp.float32)
    h = jnp.maximum(h + b_ref[...], 0.0)
    h_ref[...] = h
    u2 = jnp.dot(h.astype(jnp.bfloat16), n2_ref[...],
                 preferred_element_type=jnp.float32)
    u2_ref[...] = u2
    u2s_ref[...] = dis_ref[...] * u2


def _l2head(t2, hB, w2, b, n2, dis, rb=632):
    return pl.pallas_call(
        _l2head_body,
        grid=(NP // rb,),
        in_specs=[
            pl.BlockSpec((rb, 256), lambda i: (i, 0)),
            pl.BlockSpec((rb, 2048), lambda i: (i, 0)),
            pl.BlockSpec((256, 2048), lambda i: (0, 0)),
            pl.BlockSpec((1, 2048), lambda i: (0, 0)),
            pl.BlockSpec((2048, 1024), lambda i: (0, 0)),
            pl.BlockSpec((rb, 1), lambda i: (i, 0)),
        ],
        out_specs=[
            pl.BlockSpec((rb, 2048), lambda i: (i, 0)),
            pl.BlockSpec((rb, 1024), lambda i: (i, 0)),
            pl.BlockSpec((rb, 1024), lambda i: (i, 0)),
        ],
        out_shape=[
            jax.ShapeDtypeStruct((NP, 2048), jnp.float32),
            jax.ShapeDtypeStruct((NP, 1024), jnp.float32),
            jax.ShapeDtypeStruct((NP, 1024), jnp.float32),
        ],
        compiler_params=pltpu.CompilerParams(
            dimension_semantics=("parallel",)),
    )(t2, hB, w2, b, n2, dis)


def _u2_body(nk, x_ref, w_ref, dis_ref, u2_ref, u2s_ref):
    k = pl.program_id(2)

    @pl.when(k == 0)
    def _():
        u2_ref[...] = jnp.zeros_like(u2_ref)

    u2_ref[...] += jnp.dot(x_ref[...].astype(jnp.bfloat16), w_ref[...],
                           preferred_element_type=jnp.float32)

    @pl.when(k == nk - 1)
    def _():
        u2s_ref[...] = dis_ref[...] * u2_ref[...]


def _mm_u2(x, w, dis, rb=1264, cb=512, kb=512):
    m, kk = x.shape
    _, n = w.shape
    nk = kk // kb
    return pl.pallas_call(
        functools.partial(_u2_body, nk),
        grid=(m // rb, n // cb, nk),
        in_specs=[
            pl.BlockSpec((rb, kb), lambda i, j, k: (i, k)),
            pl.BlockSpec((kb, cb), lambda i, j, k: (k, j)),
            pl.BlockSpec((rb, 1), lambda i, j, k: (i, 0)),
        ],
        out_specs=[pl.BlockSpec((rb, cb), lambda i, j, k: (i, j))] * 2,
        out_shape=[jax.ShapeDtypeStruct((m, n), jnp.float32)] * 2,
        compiler_params=pltpu.CompilerParams(
            dimension_semantics=("parallel", "parallel", "arbitrary")),
    )(x, w, dis)


def _mm_body(nk, x_ref, w_ref, b_ref, o_ref):
    k = pl.program_id(2)

    @pl.when(k == 0)
    def _():
        o_ref[...] = jnp.zeros_like(o_ref)

    o_ref[...] += jnp.dot(x_ref[...].astype(jnp.bfloat16), w_ref[...],
                          preferred_element_type=jnp.float32)

    @pl.when(k == nk - 1)
    def _():
        o_ref[...] += b_ref[...]


def _mm(x, w, b, rb=1264, cb=512, kb=512):
    m, kk = x.shape
    _, n = w.shape
    nk = kk // kb
    return pl.pallas_call(
        functools.partial(_mm_body, nk),
        grid=(m // rb, n // cb, nk),
        in_specs=[
            pl.BlockSpec((rb, kb), lambda i, j, k: (i, k)),
            pl.BlockSpec((kb, cb), lambda i, j, k: (k, j)),
            pl.BlockSpec((1, cb), lambda i, j, k: (0, j)),
        ],
        out_specs=pl.BlockSpec((rb, cb), lambda i, j, k: (i, j)),
        out_shape=jax.ShapeDtypeStruct((m, n), jnp.float32),
        compiler_params=pltpu.CompilerParams(
            dimension_semantics=("parallel", "parallel", "arbitrary")),
    )(x, w, b)


def _band(w, in_ch, out_ch):
    li = jnp.arange(128)[:, None]
    lo = jnp.arange(128)[None, :]
    kk = li - lo + 8
    valid = (kk >= 0) & (kk <= 8)
    bm = w[:, :, jnp.clip(kk, 0, 8)]
    bm = jnp.where(valid[None, None], bm, 0.0)
    return bm.transpose(1, 2, 0, 3).reshape(in_ch * 128, out_ch * 128)


def kernel(x, edge_index, W1_0, W1_1, W1_2, b1_0, b1_1, b1_2,
           W2_0, W2_1, W2_2, b2_0, b2_1, b2_2, bias_1, bias_2):
    row = jnp.concatenate([edge_index[0], jnp.full((EP - E,), N, jnp.int32)])
    col = jnp.concatenate([edge_index[1], jnp.full((EP - E,), N, jnp.int32)])

    M0 = _band(W1_0, 2, 16)
    M1 = _band(W1_1, 2, 16)
    M2 = _band(W1_2, 2, 16)
    N0 = _band(W2_0, 16, 8)
    N1 = _band(W2_1, 16, 8)
    N2 = _band(W2_2, 16, 8)
    b1f = jnp.repeat(b1_0 + b1_1 + b1_2, 128)[None, :] + bias_1
    b2f = jnp.repeat(b2_0 + b2_1 + b2_2, 128)[None, :] + bias_2

    xpad = jnp.concatenate([x, jnp.zeros((NP - N, 256), jnp.float32)])
    const = jnp.stack([jnp.zeros((128, 128), jnp.float32),
                       jnp.ones((128, 128), jnp.float32)])

    rowp2d, rdeg2d = _remap(row.reshape(EP // 128, 128),
                            col.reshape(EP // 128, 128))
    ix_main = jnp.stack([rowp2d.reshape(NS, NB, 128),
                         col.reshape(NS, NB, 128)], axis=2)
    rdeg3 = rdeg2d.reshape(NS, NB, 128)
    ix_deg = jnp.stack([rdeg3, rdeg3], axis=2)

    deg_s = _sc_matvec(ix_deg, const, [], deg_mode=True)[0]
    dis = _reduce_dis(deg_s)

    bf = jnp.bfloat16
    xs = _ew_call(_scale_body, [xpad, dis], 1, 256)[0]
    S1 = _scatter_sum(ix_main, const, xs)
    hA = _mm(xpad, (M0 - M2).astype(bf),
             jnp.zeros((1, 2048), jnp.float32), kb=256)
    t1, t1s = _ew_call(_t1_body, [xpad, S1, dis], 2, 256)
    S2 = _scatter_sum(ix_main, const, t1s)
    hB = _mm_acc(t1, M1.astype(bf), hA)
    t2 = _ew_call(_t2_body, [t1, S2, dis], 1, 256)[0]

    h, u2, u2s = _l2head(t2, hB, (2.0 * M2).astype(bf), b1f,
                         N2.astype(bf), dis)
    S3 = _scatter_sum(ix_main, const, u2s)
    u1 = _mm(h, N1.astype(bf), jnp.zeros((1, 1024), jnp.float32))
    s, ss = _ew_call(_s_body, [u1, u2, S3, dis], 2, 1024)
    S4 = _scatter_sum(ix_main, const, ss)
    a = _mm(h, (N0 - N2).astype(bf), b2f)
    out = _ew_call(_out_body, [a, s, S4, dis], 1, 1024)[0]
    return out[:N]
